# baseline (speedup 1.0000x reference)
import functools
import math

import jax
import jax.numpy as jnp
from jax import lax
from jax.experimental import pallas as pl
from jax.experimental.pallas import tpu as pltpu
from jax.experimental.pallas import tpu_sc as plsc

N_EMBD = 384
N_HEAD = 8
N_KV = 2
HEAD = N_EMBD // N_HEAD
R = 4
E = 4
CAP_F = 1.25
LORA_SCALE = 1.0 / R
HID = 4 * N_EMBD

SC_CORES = 2
SC_SUBCORES = 16
SC_WORKERS = SC_CORES * SC_SUBCORES
LANES = 16

def _sc_mesh():
    return plsc.VectorSubcoreMesh(
        core_axis_name="c", subcore_axis_name="s",
        num_cores=SC_CORES, num_subcores=SC_SUBCORES)


def _fold_body(w_ref, a_ref, b_ref, o_ref):
    o_ref[0] = w_ref[0] + LORA_SCALE * jnp.dot(
        a_ref[0], b_ref[0], preferred_element_type=jnp.float32)


def _fold(w_t, a_t, b_t):
    g, m, n = w_t.shape
    r = a_t.shape[-1]
    return pl.pallas_call(
        _fold_body,
        grid=(g,),
        in_specs=[
            pl.BlockSpec((1, m, n), lambda i: (i, 0, 0)),
            pl.BlockSpec((1, m, r), lambda i: (i, 0, 0)),
            pl.BlockSpec((1, r, n), lambda i: (i, 0, 0)),
        ],
        out_specs=pl.BlockSpec((1, m, n), lambda i: (i, 0, 0)),
        out_shape=jax.ShapeDtypeStruct((g, m, n), jnp.float32),
    )(w_t, a_t, b_t)


def _rope_body(c_ref, s_ref):
    t = c_ref.shape[0]
    pos = lax.broadcasted_iota(jnp.int32, (t, HEAD // 2), 0).astype(jnp.float32)
    j = lax.broadcasted_iota(jnp.int32, (t, HEAD // 2), 1).astype(jnp.float32)
    inv_freq = jnp.exp(j * (-2.0 * math.log(10000.0) / HEAD))
    ang = pos * inv_freq
    c_ref[...] = jnp.cos(ang)
    s_ref[...] = jnp.sin(ang)


def _rope_cache(t):
    return pl.pallas_call(
        _rope_body,
        out_shape=(jax.ShapeDtypeStruct((t, HEAD // 2), jnp.float32),
                   jax.ShapeDtypeStruct((t, HEAD // 2), jnp.float32)),
    )()


def _qkv_body(x_ref, g_ref, b_ref, w_ref, c_ref, s_ref, o_ref):
    x = x_ref[...]
    m = jnp.mean(x, axis=-1, keepdims=True)
    v = jnp.mean((x - m) ** 2, axis=-1, keepdims=True)
    xn = (x - m) / jnp.sqrt(v + 1e-5) * g_ref[...] + b_ref[...]
    qkv = jnp.dot(xn, w_ref[...], preferred_element_type=jnp.float32)
    lane = lax.broadcasted_iota(jnp.int32, qkv.shape, 1)
    even = (lane % 2) == 0
    nl = qkv.shape[1]
    rot = jnp.where(even, pltpu.roll(qkv, nl - 1, 1), pltpu.roll(qkv, 1, 1))
    o_ref[...] = qkv * c_ref[...] + rot * s_ref[...]


def _qkv(x2d, g, b, w_t, c_full, s_full, blk):
    bt = x2d.shape[0]
    nq = N_HEAD * HEAD + 2 * N_KV * HEAD
    return pl.pallas_call(
        _qkv_body,
        grid=(bt // blk,),
        in_specs=[
            pl.BlockSpec((blk, N_EMBD), lambda i: (i, 0)),
            pl.BlockSpec((1, N_EMBD), lambda i: (0, 0)),
            pl.BlockSpec((1, N_EMBD), lambda i: (0, 0)),
            pl.BlockSpec((N_EMBD, nq), lambda i: (0, 0)),
            pl.BlockSpec((blk, nq), lambda i: (i, 0)),
            pl.BlockSpec((blk, nq), lambda i: (i, 0)),
        ],
        out_specs=pl.BlockSpec((blk, nq), lambda i: (i, 0)),
        out_shape=jax.ShapeDtypeStruct((bt, nq), jnp.float32),
    )(x2d, g, b, w_t, c_full, s_full)


def _flash_body(q_ref, kt_ref, v_ref, o_ref, m_ref, l_ref, acc_ref,
                *, blk_q, blk_k, nk):
    qi = pl.program_id(2)
    ki = pl.program_id(3)

    @pl.when(ki == 0)
    def _():
        m_ref[...] = jnp.full_like(m_ref, -1e30)
        l_ref[...] = jnp.zeros_like(l_ref)
        acc_ref[...] = jnp.zeros_like(acc_ref)

    @pl.when(ki * blk_k <= qi * blk_q + blk_q - 1)
    def _():
        q = q_ref[0, 0]
        kt = kt_ref[0, 0]
        s = jnp.dot(q, kt, preferred_element_type=jnp.float32)
        s = s * (1.0 / math.sqrt(HEAD))
        qpos = qi * blk_q + lax.broadcasted_iota(jnp.int32, s.shape, 0)
        kpos = ki * blk_k + lax.broadcasted_iota(jnp.int32, s.shape, 1)
        s = jnp.where(kpos <= qpos, s, -1e30)
        m_prev = m_ref[:, 0:1]
        m_cur = jnp.max(s, axis=-1, keepdims=True)
        m_new = jnp.maximum(m_prev, m_cur)
        alpha = jnp.exp(m_prev - m_new)
        p = jnp.exp(s - m_new)
        l_ref[:, 0:1] = l_ref[:, 0:1] * alpha + jnp.sum(p, axis=-1,
                                                        keepdims=True)
        m_ref[:, 0:1] = m_new
        acc_ref[...] = acc_ref[...] * alpha + jnp.dot(
            p, v_ref[0, 0], preferred_element_type=jnp.float32)

    @pl.when(ki == nk - 1)
    def _():
        o_ref[0, 0] = acc_ref[...] / l_ref[:, 0:1]


def _flash(q, kt, v, blk_q, blk_k):
    b, h, t, d = q.shape
    nq, nk = t // blk_q, t // blk_k
    rep = N_HEAD // N_KV
    return pl.pallas_call(
        functools.partial(_flash_body, blk_q=blk_q, blk_k=blk_k, nk=nk),
        grid=(b, h, nq, nk),
        in_specs=[
            pl.BlockSpec((1, 1, blk_q, d),
                         lambda b_, h_, qi, ki: (b_, h_, qi, 0)),
            pl.BlockSpec((1, 1, d, blk_k),
                         lambda b_, h_, qi, ki: (b_, h_ // rep, 0, ki)),
            pl.BlockSpec((1, 1, blk_k, d),
                         lambda b_, h_, qi, ki: (b_, h_ // rep, ki, 0)),
        ],
        out_specs=pl.BlockSpec((1, 1, blk_q, d),
                               lambda b_, h_, qi, ki: (b_, h_, qi, 0)),
        out_shape=jax.ShapeDtypeStruct((b, h, t, d), jnp.float32),
        scratch_shapes=[
            pltpu.VMEM((blk_q, 128), jnp.float32),
            pltpu.VMEM((blk_q, 128), jnp.float32),
            pltpu.VMEM((blk_q, d), jnp.float32),
        ],
    )(q, kt, v)


def _oproj_body(a_ref, x_ref, w_ref, g_ref, b_ref, h_ref, xn_ref):
    h = jnp.dot(a_ref[...], w_ref[...],
                preferred_element_type=jnp.float32) + x_ref[...]
    h_ref[...] = h
    m = jnp.mean(h, axis=-1, keepdims=True)
    v = jnp.mean((h - m) ** 2, axis=-1, keepdims=True)
    xn_ref[...] = (h - m) / jnp.sqrt(v + 1e-5) * g_ref[...] + b_ref[...]


def _oproj(a2d, x2d, wo_t, g, b, blk):
    bt = a2d.shape[0]
    return pl.pallas_call(
        _oproj_body,
        grid=(bt // blk,),
        in_specs=[
            pl.BlockSpec((blk, N_EMBD), lambda i: (i, 0)),
            pl.BlockSpec((blk, N_EMBD), lambda i: (i, 0)),
            pl.BlockSpec((N_EMBD, N_EMBD), lambda i: (0, 0)),
            pl.BlockSpec((1, N_EMBD), lambda i: (0, 0)),
            pl.BlockSpec((1, N_EMBD), lambda i: (0, 0)),
        ],
        out_specs=(pl.BlockSpec((blk, N_EMBD), lambda i: (i, 0)),
                   pl.BlockSpec((blk, N_EMBD), lambda i: (i, 0))),
        out_shape=(jax.ShapeDtypeStruct((bt, N_EMBD), jnp.float32),
                   jax.ShapeDtypeStruct((bt, N_EMBD), jnp.float32)),
    )(a2d, x2d, wo_t, g, b)


def _router_body(xn_ref, wr_ref, br_ref, dest_ref, gate_ref, aux_ref,
                 cnt_ref, psum_ref, lsum_ref, *, blk, nblk, cap, bt):
    i = pl.program_id(0)

    @pl.when(i == 0)
    def _():
        cnt_ref[...] = jnp.zeros_like(cnt_ref)
        psum_ref[...] = jnp.zeros_like(psum_ref)
        lsum_ref[...] = jnp.zeros_like(lsum_ref)

    xn = xn_ref[...]
    logits = jnp.dot(xn, wr_ref[...],
                     preferred_element_type=jnp.float32) + br_ref[...]
    mx = jnp.max(logits, axis=-1, keepdims=True)
    ex = jnp.exp(logits - mx)
    probs = ex / jnp.sum(ex, axis=-1, keepdims=True)
    top_v = jnp.max(probs, axis=-1, keepdims=True)
    lane = lax.broadcasted_iota(jnp.int32, probs.shape, 1)
    idx = jnp.min(jnp.where(probs >= top_v, lane, E), axis=-1,
                  keepdims=True)
    onehot = (lane == idx).astype(jnp.float32)
    row = lax.broadcasted_iota(jnp.int32, (blk, blk), 0)
    col = lax.broadcasted_iota(jnp.int32, (blk, blk), 1)
    tri = (row >= col).astype(jnp.float32)
    csum = jnp.dot(tri, onehot, preferred_element_type=jnp.float32)
    pos = csum - 1.0 + cnt_ref[0:1, 0:E]
    disp = onehot * (pos < cap).astype(jnp.float32)
    disp_tok = jnp.sum(disp, axis=-1, keepdims=True)
    pos_tok = jnp.sum(disp * pos, axis=-1, keepdims=True)
    rowid = i * blk + lax.broadcasted_iota(jnp.int32, (blk, 1), 0)
    dest_hit = idx * cap + pos_tok.astype(jnp.int32)
    trash = E * cap + rowid // (bt // SC_WORKERS)
    dest_ref[...] = jnp.where(disp_tok > 0.0, dest_hit, trash)
    gate_ref[...] = top_v * disp_tok
    cnt_ref[0:1, 0:E] = cnt_ref[0:1, 0:E] + jnp.sum(onehot, axis=0,
                                                    keepdims=True)
    psum_ref[0:1, 0:E] = psum_ref[0:1, 0:E] + jnp.sum(probs, axis=0,
                                                      keepdims=True)
    lsum_ref[0:1, 0:E] = lsum_ref[0:1, 0:E] + jnp.sum(disp, axis=0,
                                                      keepdims=True)

    @pl.when(i == nblk - 1)
    def _():
        prod = psum_ref[0:1, 0:E] * lsum_ref[0:1, 0:E]
        aux_ref[...] = jnp.sum(prod, axis=-1, keepdims=True) \
            * (float(E) / (float(bt) * float(bt)))


def _router(xn2, wr_t, br, cap, blk):
    bt = xn2.shape[0]
    nblk = bt // blk
    body = functools.partial(_router_body, blk=blk, nblk=nblk, cap=cap, bt=bt)
    return pl.pallas_call(
        body,
        grid=(nblk,),
        in_specs=[
            pl.BlockSpec((blk, N_EMBD), lambda i: (i, 0)),
            pl.BlockSpec((N_EMBD, E), lambda i: (0, 0)),
            pl.BlockSpec((1, E), lambda i: (0, 0)),
        ],
        out_specs=(pl.BlockSpec((blk, 1), lambda i: (i, 0)),
                   pl.BlockSpec((blk, 1), lambda i: (i, 0)),
                   pl.BlockSpec((1, 1), lambda i: (0, 0))),
        out_shape=(jax.ShapeDtypeStruct((bt, 1), jnp.int32),
                   jax.ShapeDtypeStruct((bt, 1), jnp.float32),
                   jax.ShapeDtypeStruct((1, 1), jnp.float32)),
        scratch_shapes=[
            pltpu.VMEM((1, 128), jnp.float32),
            pltpu.VMEM((1, 128), jnp.float32),
            pltpu.VMEM((1, 128), jnp.float32),
        ],
    )(xn2, wr_t, br)


def _sc_scatter(xn2, dest, nslot):
    bt, c = xn2.shape
    tpw = bt // SC_WORKERS

    @functools.partial(
        pl.kernel,
        out_type=jax.ShapeDtypeStruct((nslot, c), jnp.float32),
        mesh=_sc_mesh(),
        scratch_types=[
            pltpu.VMEM((tpw,), jnp.int32),
            pltpu.VMEM((tpw, c), jnp.float32),
            pltpu.SemaphoreType.DMA,
        ],
    )
    def k(xn_hbm, dest_hbm, buf_hbm, idx_v, rows_v, sem):
        wid = lax.axis_index("s") * SC_CORES + lax.axis_index("c")
        base = wid * tpw
        pltpu.sync_copy(dest_hbm.at[pl.ds(base, tpw)], idx_v)
        pltpu.sync_copy(xn_hbm.at[pl.ds(base, tpw)], rows_v)
        pltpu.async_copy(rows_v, buf_hbm.at[idx_v], sem).wait()

    return k(xn2, dest)


def _sc_gather(ybuf, dest, gate, h2d):
    bt, c = h2d.shape
    tpw = bt // SC_WORKERS
    nch = c // LANES

    @functools.partial(
        pl.kernel,
        out_type=jax.ShapeDtypeStruct((bt, c), jnp.float32),
        mesh=_sc_mesh(),
        scratch_types=[
            pltpu.VMEM((tpw,), jnp.int32),
            pltpu.VMEM((tpw,), jnp.float32),
            pltpu.VMEM((tpw, c), jnp.float32),
            pltpu.VMEM((tpw, c), jnp.float32),
            pltpu.SemaphoreType.DMA,
        ],
    )
    def k(ybuf_hbm, dest_hbm, gate_hbm, h_hbm, out_hbm,
          idx_v, gate_v, y_v, h_v, sem):
        wid = lax.axis_index("s") * SC_CORES + lax.axis_index("c")
        base = wid * tpw
        pltpu.sync_copy(dest_hbm.at[pl.ds(base, tpw)], idx_v)
        pltpu.sync_copy(gate_hbm.at[pl.ds(base, tpw)], gate_v)
        pltpu.sync_copy(h_hbm.at[pl.ds(base, tpw)], h_v)
        pltpu.async_copy(ybuf_hbm.at[idx_v], y_v, sem).wait()

        def body(i, carry):
            g = gate_v[pl.ds(i, 1)][0]
            for ch in range(nch):
                sl = pl.ds(ch * LANES, LANES)
                y = y_v[i, sl]
                safe = jnp.where(g > 0.0, y, jnp.zeros_like(y))
                h_v[i, sl] = h_v[i, sl] + safe * g
            return carry

        lax.fori_loop(0, tpw, body, 0)
        pltpu.sync_copy(h_v, out_hbm.at[pl.ds(base, tpw)])

    return k(ybuf, dest, gate, h2d)


def _mlp_body(x_ref, w1_ref, w2_ref, y_ref):
    h = jnp.dot(x_ref[...], w1_ref[0], preferred_element_type=jnp.float32)
    h = jnp.maximum(h, 0.0)
    y_ref[...] = jnp.dot(h, w2_ref[0], preferred_element_type=jnp.float32)


def _mlp(buf, w1_t, w2_t, cap, nslot, blk):
    nblk = cap // blk
    return pl.pallas_call(
        _mlp_body,
        grid=(E, nblk),
        in_specs=[
            pl.BlockSpec((blk, N_EMBD), lambda e, i: (e * nblk + i, 0)),
            pl.BlockSpec((1, N_EMBD, HID), lambda e, i: (e, 0, 0)),
            pl.BlockSpec((1, HID, N_EMBD), lambda e, i: (e, 0, 0)),
        ],
        out_specs=pl.BlockSpec((blk, N_EMBD), lambda e, i: (e * nblk + i, 0)),
        out_shape=jax.ShapeDtypeStruct((nslot, N_EMBD), jnp.float32),
    )(buf, w1_t, w2_t)


def kernel(x, adapter_id, params):
    p = params
    b, t, c = x.shape
    bt = b * t
    cap = int(math.ceil(CAP_F * bt / E))
    nslot = E * cap + SC_WORKERS
    aid = jnp.asarray(adapter_id).astype(jnp.int32)

    aq, ak, av, ao = p['Aq'][aid], p['Ak'][aid], p['Av'][aid], p['Ao'][aid]
    bq, bk, bv, bo = p['Bq'][aid], p['Bk'][aid], p['Bv'][aid], p['Bo'][aid]
    w_all_t = jnp.concatenate(
        [p['Wq'].T, p['Wk'].T, p['Wv'].T, p['Wo'].T], axis=1)
    a_all_t = jnp.concatenate([aq.T, ak.T, av.T, ao.T], axis=1)
    nq = N_HEAD * HEAD
    nkv = N_KV * HEAD
    b_bd_t = jnp.zeros((4 * R, 2 * nkv + 2 * nq), jnp.float32)
    b_bd_t = b_bd_t.at[0:R, 0:nq].set(bq.T)
    b_bd_t = b_bd_t.at[R:2 * R, nq:nq + nkv].set(bk.T)
    b_bd_t = b_bd_t.at[2 * R:3 * R, nq + nkv:nq + 2 * nkv].set(bv.T)
    b_bd_t = b_bd_t.at[3 * R:4 * R, nq + 2 * nkv:].set(bo.T)
    qkvo_t = _fold(w_all_t[None], a_all_t[None], b_bd_t[None])[0]
    wqkv_t = qkvo_t[:, :nq + 2 * nkv]
    wo_t = qkvo_t[:, nq + 2 * nkv:]

    w1_t = _fold(p['W1'].transpose(0, 2, 1),
                 p['A1'][:, aid].transpose(0, 2, 1),
                 p['B1'][:, aid].transpose(0, 2, 1))
    w2_t = _fold(p['W2'].transpose(0, 2, 1),
                 p['A2'][:, aid].transpose(0, 2, 1),
                 p['B2'][:, aid].transpose(0, 2, 1))

    c24, s24 = _rope_cache(t)
    cc = jnp.repeat(c24, 2, axis=1)
    ss = jnp.repeat(s24, 2, axis=1)
    sgn = jnp.tile(jnp.array([-1.0, 1.0], jnp.float32), HEAD // 2)[None]
    n_rot = N_HEAD + N_KV
    c_full = jnp.concatenate(
        [jnp.tile(cc, (1, n_rot)), jnp.ones((t, nkv), jnp.float32)], axis=1)
    s_full = jnp.concatenate(
        [jnp.tile(ss * sgn, (1, n_rot)), jnp.zeros((t, nkv), jnp.float32)],
        axis=1)
    c_full = jnp.tile(c_full, (b, 1))
    s_full = jnp.tile(s_full, (b, 1))

    x2d = x.reshape(bt, c)
    qkv = _qkv(x2d, p['ln1_g'][None], p['ln1_b'][None], wqkv_t,
               c_full, s_full, blk=256)
    q = qkv[:, :nq].reshape(b, t, N_HEAD, HEAD).transpose(0, 2, 1, 3)
    kt = qkv[:, nq:nq + nkv].reshape(b, t, N_KV, HEAD).transpose(0, 2, 3, 1)
    v = qkv[:, nq + nkv:].reshape(b, t, N_KV, HEAD).transpose(0, 2, 1, 3)
    attn = _flash(q, kt, v, blk_q=256, blk_k=512)
    attn2d = attn.transpose(0, 2, 1, 3).reshape(bt, c)
    h2d, xn2 = _oproj(attn2d, x2d, wo_t, p['ln2_g'][None], p['ln2_b'][None],
                      blk=256)

    dest, gate, aux = _router(xn2, p['Wr'].T, p['br'][None], cap, blk=512)
    dest = dest.reshape(bt)
    gate = gate.reshape(bt)
    buf = _sc_scatter(xn2, dest, nslot)
    ybuf = _mlp(buf, w1_t, w2_t, cap, nslot, blk=256)
    out2d = _sc_gather(ybuf, dest, gate, h2d)
    return out2d.reshape(b, t, c), aux.reshape(())


# speedup vs baseline: 1.3577x; 1.3577x over previous
import functools
import math

import jax
import jax.numpy as jnp
from jax import lax
from jax.experimental import pallas as pl
from jax.experimental.pallas import tpu as pltpu
from jax.experimental.pallas import tpu_sc as plsc

N_EMBD = 384
N_HEAD = 8
N_KV = 2
HEAD = N_EMBD // N_HEAD
R = 4
E = 4
CAP_F = 1.25
LORA_SCALE = 1.0 / R
HID = 4 * N_EMBD
NQ = N_HEAD * HEAD
NKV = N_KV * HEAD
QKV_W = NQ + 2 * NKV

SC_CORES = 2
SC_SUBCORES = 16
SC_WORKERS = SC_CORES * SC_SUBCORES


def _sc_mesh():
    return plsc.VectorSubcoreMesh(
        core_axis_name="c", subcore_axis_name="s",
        num_cores=SC_CORES, num_subcores=SC_SUBCORES)


def _dot_t(x, w):
    return lax.dot_general(x, w, (((1,), (1,)), ((), ())),
                           preferred_element_type=jnp.float32)


def _fold_body(w_ref, b_ref, a_ref, o_ref):
    o_ref[0] = w_ref[0] + LORA_SCALE * jnp.dot(
        b_ref[0], a_ref[0], preferred_element_type=jnp.float32)


def _fold(w, b, a):
    g, m, n = w.shape
    r = b.shape[-1]
    return pl.pallas_call(
        _fold_body,
        grid=(g,),
        in_specs=[
            pl.BlockSpec((1, m, n), lambda i: (i, 0, 0)),
            pl.BlockSpec((1, m, r), lambda i: (i, 0, 0)),
            pl.BlockSpec((1, r, n), lambda i: (i, 0, 0)),
        ],
        out_specs=pl.BlockSpec((1, m, n), lambda i: (i, 0, 0)),
        out_shape=jax.ShapeDtypeStruct((g, m, n), jnp.float32),
    )(w, b, a)


def _rope_body(c_ref, s_ref):
    t = c_ref.shape[0]
    pos = lax.broadcasted_iota(jnp.int32, (t, HEAD // 2), 0).astype(jnp.float32)
    j = lax.broadcasted_iota(jnp.int32, (t, HEAD // 2), 1).astype(jnp.float32)
    ang = pos * jnp.exp(j * (-2.0 * math.log(10000.0) / HEAD))
    c24 = jnp.cos(ang)
    s24 = jnp.sin(ang)
    jr = lax.broadcasted_iota(jnp.int32, (HEAD // 2, QKV_W), 0)
    lc = lax.broadcasted_iota(jnp.int32, (HEAD // 2, QKV_W), 1)
    rot = lc < (NQ + NKV)
    sel = (((lc % HEAD) // 2) == jr) & rot
    m = sel.astype(jnp.float32)
    sgn = jnp.where((lc % 2) == 0, -1.0, 1.0)
    vlane = jnp.where(rot, 0.0, 1.0)[0:1]
    c_ref[...] = jnp.dot(c24, m, preferred_element_type=jnp.float32) + vlane
    s_ref[...] = jnp.dot(s24, m * sgn, preferred_element_type=jnp.float32)


def _rope_tables(t):
    return pl.pallas_call(
        _rope_body,
        out_shape=(jax.ShapeDtypeStruct((t, QKV_W), jnp.float32),
                   jax.ShapeDtypeStruct((t, QKV_W), jnp.float32)),
    )()


def _qkv_body(x_ref, g_ref, b_ref, w_ref, c_ref, s_ref, o_ref):
    x = x_ref[...]
    m = jnp.mean(x, axis=-1, keepdims=True)
    v = jnp.mean((x - m) ** 2, axis=-1, keepdims=True)
    xn = (x - m) / jnp.sqrt(v + 1e-5) * g_ref[...] + b_ref[...]
    qkv = _dot_t(xn, w_ref[...])
    lane = lax.broadcasted_iota(jnp.int32, qkv.shape, 1)
    even = (lane % 2) == 0
    nl = qkv.shape[1]
    rot = jnp.where(even, pltpu.roll(qkv, nl - 1, 1), pltpu.roll(qkv, 1, 1))
    qkv = qkv * c_ref[...] + rot * s_ref[...]
    for j in range(QKV_W // HEAD):
        o_ref[0, j] = qkv[:, j * HEAD:(j + 1) * HEAD]


def _qkv(x2d, g, b, w, c_tab, s_tab, bb, t, blk):
    bt = x2d.shape[0]
    tb = t // blk
    nh = QKV_W // HEAD
    return pl.pallas_call(
        _qkv_body,
        grid=(bt // blk,),
        in_specs=[
            pl.BlockSpec((blk, N_EMBD), lambda i: (i, 0)),
            pl.BlockSpec((1, N_EMBD), lambda i: (0, 0)),
            pl.BlockSpec((1, N_EMBD), lambda i: (0, 0)),
            pl.BlockSpec((QKV_W, N_EMBD), lambda i: (0, 0)),
            pl.BlockSpec((blk, QKV_W), lambda i: (i % tb, 0)),
            pl.BlockSpec((blk, QKV_W), lambda i: (i % tb, 0)),
        ],
        out_specs=pl.BlockSpec((1, nh, blk, HEAD),
                               lambda i: (i // tb, 0, i % tb, 0)),
        out_shape=jax.ShapeDtypeStruct((bb, nh, t, HEAD), jnp.float32),
    )(x2d, g, b, w, c_tab, s_tab)


def _flash_body(q_ref, k_ref, v_ref, o_ref, m_ref, l_ref, acc_ref,
                *, blk_q, blk_k, nk):
    qi = pl.program_id(2)
    ki = pl.program_id(3)

    @pl.when(ki == 0)
    def _():
        m_ref[...] = jnp.full_like(m_ref, -1e30)
        l_ref[...] = jnp.zeros_like(l_ref)
        acc_ref[...] = jnp.zeros_like(acc_ref)

    @pl.when(ki * blk_k <= qi * blk_q + blk_q - 1)
    def _():
        q = q_ref[0, 0]
        s = lax.dot_general(q, k_ref[0, 0], (((1,), (1,)), ((), ())),
                            preferred_element_type=jnp.float32)
        s = s * (1.0 / math.sqrt(HEAD))
        qpos = qi * blk_q + lax.broadcasted_iota(jnp.int32, s.shape, 0)
        kpos = ki * blk_k + lax.broadcasted_iota(jnp.int32, s.shape, 1)
        s = jnp.where(kpos <= qpos, s, -1e30)
        m_prev = m_ref[:, 0:1]
        m_cur = jnp.max(s, axis=-1, keepdims=True)
        m_new = jnp.maximum(m_prev, m_cur)
        alpha = jnp.exp(m_prev - m_new)
        p = jnp.exp(s - m_new)
        l_ref[:, 0:1] = l_ref[:, 0:1] * alpha + jnp.sum(p, axis=-1,
                                                        keepdims=True)
        m_ref[:, 0:1] = m_new
        acc_ref[...] = acc_ref[...] * alpha + jnp.dot(
            p, v_ref[0, 0], preferred_element_type=jnp.float32)

    @pl.when(ki == nk - 1)
    def _():
        o_ref[0, 0] = acc_ref[...] / l_ref[:, 0:1]


def _flash(qkvh, blk_q, blk_k):
    bb, _, t, d = qkvh.shape
    nq, nk = t // blk_q, t // blk_k
    rep = N_HEAD // N_KV
    kcol = NQ // HEAD
    vcol = (NQ + NKV) // HEAD
    return pl.pallas_call(
        functools.partial(_flash_body, blk_q=blk_q, blk_k=blk_k, nk=nk),
        grid=(bb, N_HEAD, nq, nk),
        in_specs=[
            pl.BlockSpec((1, 1, blk_q, d),
                         lambda b_, h_, qi, ki: (b_, h_, qi, 0)),
            pl.BlockSpec((1, 1, blk_k, d),
                         lambda b_, h_, qi, ki: (b_, kcol + h_ // rep, ki, 0)),
            pl.BlockSpec((1, 1, blk_k, d),
                         lambda b_, h_, qi, ki: (b_, vcol + h_ // rep, ki, 0)),
        ],
        out_specs=pl.BlockSpec((1, 1, blk_q, d),
                               lambda b_, h_, qi, ki: (b_, h_, qi, 0)),
        out_shape=jax.ShapeDtypeStruct((bb, N_HEAD, t, d), jnp.float32),
        scratch_shapes=[
            pltpu.VMEM((blk_q, 128), jnp.float32),
            pltpu.VMEM((blk_q, 128), jnp.float32),
            pltpu.VMEM((blk_q, d), jnp.float32),
        ],
    )(qkvh, qkvh, qkvh)


def _oproj_body(a_ref, x_ref, w_ref, g_ref, b_ref, h_ref, xn_ref):
    a = jnp.concatenate([a_ref[0, j] for j in range(N_HEAD)], axis=1)
    h = _dot_t(a, w_ref[...]) + x_ref[...]
    h_ref[...] = h
    m = jnp.mean(h, axis=-1, keepdims=True)
    v = jnp.mean((h - m) ** 2, axis=-1, keepdims=True)
    xn_ref[...] = (h - m) / jnp.sqrt(v + 1e-5) * g_ref[...] + b_ref[...]


def _oproj(attnh, x2d, wo, g, b, t, blk):
    bt = x2d.shape[0]
    tb = t // blk
    return pl.pallas_call(
        _oproj_body,
        grid=(bt // blk,),
        in_specs=[
            pl.BlockSpec((1, N_HEAD, blk, HEAD),
                         lambda i: (i // tb, 0, i % tb, 0)),
            pl.BlockSpec((blk, N_EMBD), lambda i: (i, 0)),
            pl.BlockSpec((N_EMBD, N_EMBD), lambda i: (0, 0)),
            pl.BlockSpec((1, N_EMBD), lambda i: (0, 0)),
            pl.BlockSpec((1, N_EMBD), lambda i: (0, 0)),
        ],
        out_specs=(pl.BlockSpec((blk, N_EMBD), lambda i: (i, 0)),
                   pl.BlockSpec((blk, N_EMBD), lambda i: (i, 0))),
        out_shape=(jax.ShapeDtypeStruct((bt, N_EMBD), jnp.float32),
                   jax.ShapeDtypeStruct((bt, N_EMBD), jnp.float32)),
    )(attnh, x2d, wo, g, b)


def _router_body(xn_ref, wr_ref, br_ref, dest_ref, gate_ref, aux_ref,
                 cnt_ref, psum_ref, lsum_ref, *, blk, nblk, cap, bt):
    i = pl.program_id(0)

    @pl.when(i == 0)
    def _():
        cnt_ref[...] = jnp.zeros_like(cnt_ref)
        psum_ref[...] = jnp.zeros_like(psum_ref)
        lsum_ref[...] = jnp.zeros_like(lsum_ref)

    xn = xn_ref[...]
    logits = _dot_t(xn, wr_ref[...]) + br_ref[...]
    mx = jnp.max(logits, axis=-1, keepdims=True)
    ex = jnp.exp(logits - mx)
    probs = ex / jnp.sum(ex, axis=-1, keepdims=True)
    top_v = jnp.max(probs, axis=-1, keepdims=True)
    lane = lax.broadcasted_iota(jnp.int32, probs.shape, 1)
    idx = jnp.min(jnp.where(probs >= top_v, lane, E), axis=-1,
                  keepdims=True)
    onehot = (lane == idx).astype(jnp.float32)
    row = lax.broadcasted_iota(jnp.int32, (blk, blk), 0)
    col = lax.broadcasted_iota(jnp.int32, (blk, blk), 1)
    tri = (row >= col).astype(jnp.float32)
    csum = jnp.dot(tri, onehot, preferred_element_type=jnp.float32)
    pos = csum - 1.0 + cnt_ref[0:1, 0:E]
    disp = onehot * (pos < cap).astype(jnp.float32)
    disp_tok = jnp.sum(disp, axis=-1, keepdims=True)
    pos_tok = jnp.sum(disp * pos, axis=-1, keepdims=True)
    rowid = i * blk + lax.broadcasted_iota(jnp.int32, (blk, 1), 0)
    dest_hit = idx * cap + pos_tok.astype(jnp.int32)
    trash = E * cap + rowid // (bt // SC_WORKERS)
    dest_ref[...] = jnp.where(disp_tok > 0.0, dest_hit, trash)
    gate_ref[...] = top_v * disp_tok
    cnt_ref[0:1, 0:E] = cnt_ref[0:1, 0:E] + jnp.sum(onehot, axis=0,
                                                    keepdims=True)
    psum_ref[0:1, 0:E] = psum_ref[0:1, 0:E] + jnp.sum(probs, axis=0,
                                                      keepdims=True)
    lsum_ref[0:1, 0:E] = lsum_ref[0:1, 0:E] + jnp.sum(disp, axis=0,
                                                      keepdims=True)

    @pl.when(i == nblk - 1)
    def _():
        prod = psum_ref[0:1, 0:E] * lsum_ref[0:1, 0:E]
        aux_ref[...] = jnp.sum(prod, axis=-1, keepdims=True) \
            * (float(E) / (float(bt) * float(bt)))


def _router(xn2, wr, br, cap, blk):
    bt = xn2.shape[0]
    nblk = bt // blk
    body = functools.partial(_router_body, blk=blk, nblk=nblk, cap=cap, bt=bt)
    return pl.pallas_call(
        body,
        grid=(nblk,),
        in_specs=[
            pl.BlockSpec((blk, N_EMBD), lambda i: (i, 0)),
            pl.BlockSpec((E, N_EMBD), lambda i: (0, 0)),
            pl.BlockSpec((1, E), lambda i: (0, 0)),
        ],
        out_specs=(pl.BlockSpec((blk, 1), lambda i: (i, 0)),
                   pl.BlockSpec((blk, 1), lambda i: (i, 0)),
                   pl.BlockSpec((1, 1), lambda i: (0, 0))),
        out_shape=(jax.ShapeDtypeStruct((bt, 1), jnp.int32),
                   jax.ShapeDtypeStruct((bt, 1), jnp.float32),
                   jax.ShapeDtypeStruct((1, 1), jnp.float32)),
        scratch_shapes=[
            pltpu.VMEM((1, 128), jnp.float32),
            pltpu.VMEM((1, 128), jnp.float32),
            pltpu.VMEM((1, 128), jnp.float32),
        ],
    )(xn2, wr, br)


def _sc_scatter(xn2, dest, nslot):
    bt, c = xn2.shape
    tpw = bt // SC_WORKERS

    @functools.partial(
        pl.kernel,
        out_type=jax.ShapeDtypeStruct((nslot, c), jnp.float32),
        mesh=_sc_mesh(),
        scratch_types=[
            pltpu.VMEM((tpw,), jnp.int32),
            pltpu.VMEM((tpw, c), jnp.float32),
            pltpu.SemaphoreType.DMA,
        ],
    )
    def k(xn_hbm, dest_hbm, buf_hbm, idx_v, rows_v, sem):
        wid = lax.axis_index("s") * SC_CORES + lax.axis_index("c")
        base = wid * tpw
        pltpu.sync_copy(dest_hbm.at[pl.ds(base, tpw)], idx_v)
        pltpu.sync_copy(xn_hbm.at[pl.ds(base, tpw)], rows_v)
        pltpu.async_copy(rows_v, buf_hbm.at[idx_v], sem).wait()

    return k(xn2, dest)


def _sc_gather(ybuf, dest, bt):
    c = ybuf.shape[1]
    tpw = bt // SC_WORKERS

    @functools.partial(
        pl.kernel,
        out_type=jax.ShapeDtypeStruct((bt, c), jnp.float32),
        mesh=_sc_mesh(),
        scratch_types=[
            pltpu.VMEM((tpw,), jnp.int32),
            pltpu.VMEM((tpw, c), jnp.float32),
            pltpu.SemaphoreType.DMA,
        ],
    )
    def k(ybuf_hbm, dest_hbm, out_hbm, idx_v, y_v, sem):
        wid = lax.axis_index("s") * SC_CORES + lax.axis_index("c")
        base = wid * tpw
        pltpu.sync_copy(dest_hbm.at[pl.ds(base, tpw)], idx_v)
        pltpu.async_copy(ybuf_hbm.at[idx_v], y_v, sem).wait()
        pltpu.sync_copy(y_v, out_hbm.at[pl.ds(base, tpw)])

    return k(ybuf, dest)


def _mlp_body(x_ref, w1_ref, w2_ref, y_ref):
    h = _dot_t(x_ref[...], w1_ref[0])
    h = jnp.maximum(h, 0.0)
    y_ref[...] = _dot_t(h, w2_ref[0])


def _mlp(buf, w1, w2, cap, nslot, blk):
    nblk = cap // blk
    return pl.pallas_call(
        _mlp_body,
        grid=(E, nblk),
        in_specs=[
            pl.BlockSpec((blk, N_EMBD), lambda e, i: (e * nblk + i, 0)),
            pl.BlockSpec((1, HID, N_EMBD), lambda e, i: (e, 0, 0)),
            pl.BlockSpec((1, N_EMBD, HID), lambda e, i: (e, 0, 0)),
        ],
        out_specs=pl.BlockSpec((blk, N_EMBD), lambda e, i: (e * nblk + i, 0)),
        out_shape=jax.ShapeDtypeStruct((nslot, N_EMBD), jnp.float32),
    )(buf, w1, w2)


def _epi_body(h_ref, g_ref, y_ref, o_ref):
    g = g_ref[...]
    o_ref[...] = h_ref[...] + jnp.where(g > 0.0, g * y_ref[...], 0.0)


def _epilogue(h2d, gate, ygath, blk):
    bt = h2d.shape[0]
    return pl.pallas_call(
        _epi_body,
        grid=(bt // blk,),
        in_specs=[
            pl.BlockSpec((blk, N_EMBD), lambda i: (i, 0)),
            pl.BlockSpec((blk, 1), lambda i: (i, 0)),
            pl.BlockSpec((blk, N_EMBD), lambda i: (i, 0)),
        ],
        out_specs=pl.BlockSpec((blk, N_EMBD), lambda i: (i, 0)),
        out_shape=jax.ShapeDtypeStruct((bt, N_EMBD), jnp.float32),
    )(h2d, gate, ygath)


def kernel(x, adapter_id, params):
    p = params
    b, t, c = x.shape
    bt = b * t
    cap = int(math.ceil(CAP_F * bt / E))
    nslot = E * cap + SC_WORKERS
    aid = jnp.asarray(adapter_id).astype(jnp.int32)

    w_all = jnp.concatenate([p['Wq'], p['Wk'], p['Wv'], p['Wo']], axis=0)
    a_all = jnp.concatenate(
        [p['Aq'][aid], p['Ak'][aid], p['Av'][aid], p['Ao'][aid]], axis=0)
    b_bd = jnp.zeros((2 * NQ + 2 * NKV, 4 * R), jnp.float32)
    b_bd = b_bd.at[0:NQ, 0:R].set(p['Bq'][aid])
    b_bd = b_bd.at[NQ:NQ + NKV, R:2 * R].set(p['Bk'][aid])
    b_bd = b_bd.at[NQ + NKV:NQ + 2 * NKV, 2 * R:3 * R].set(p['Bv'][aid])
    b_bd = b_bd.at[NQ + 2 * NKV:, 3 * R:].set(p['Bo'][aid])
    qkvo = _fold(w_all[None], b_bd[None], a_all[None])[0]
    wqkv = qkvo[:QKV_W]
    wo = qkvo[QKV_W:]

    w1 = _fold(p['W1'], p['B1'][:, aid], p['A1'][:, aid])
    w2 = _fold(p['W2'], p['B2'][:, aid], p['A2'][:, aid])

    c_tab, s_tab = _rope_tables(t)
    x2d = x.reshape(bt, c)
    qkvh = _qkv(x2d, p['ln1_g'][None], p['ln1_b'][None], wqkv,
                c_tab, s_tab, b, t, blk=256)
    attnh = _flash(qkvh, blk_q=512, blk_k=512)
    h2d, xn2 = _oproj(attnh, x2d, wo, p['ln2_g'][None], p['ln2_b'][None],
                      t, blk=256)

    dest, gate, aux = _router(xn2, p['Wr'], p['br'][None], cap, blk=512)
    buf = _sc_scatter(xn2, dest.reshape(bt), nslot)
    ybuf = _mlp(buf, w1, w2, cap, nslot, blk=640)
    ygath = _sc_gather(ybuf, dest.reshape(bt), bt)
    out2d = _epilogue(h2d, gate, ygath, blk=512)
    return out2d.reshape(b, t, c), aux.reshape(())


# speedup vs baseline: 1.3900x; 1.0238x over previous
import functools
import math

import jax
import jax.numpy as jnp
from jax import lax
from jax.experimental import pallas as pl
from jax.experimental.pallas import tpu as pltpu
from jax.experimental.pallas import tpu_sc as plsc

N_EMBD = 384
N_HEAD = 8
N_KV = 2
HEAD = N_EMBD // N_HEAD
R = 4
E = 4
CAP_F = 1.25
LORA_SCALE = 1.0 / R
HID = 4 * N_EMBD
NQ = N_HEAD * HEAD
NKV = N_KV * HEAD
QKV_W = NQ + 2 * NKV

SC_CORES = 2
SC_SUBCORES = 16
SC_WORKERS = SC_CORES * SC_SUBCORES


def _sc_mesh():
    return plsc.VectorSubcoreMesh(
        core_axis_name="c", subcore_axis_name="s",
        num_cores=SC_CORES, num_subcores=SC_SUBCORES)


def _dot_t(x, w):
    return lax.dot_general(x, w, (((1,), (1,)), ((), ())),
                           preferred_element_type=jnp.float32)


def _fold_body(w_ref, b_ref, a_ref, o_ref):
    eff = w_ref[0] + LORA_SCALE * jnp.dot(
        b_ref[0], a_ref[0], preferred_element_type=jnp.float32)
    o_ref[0] = eff.astype(o_ref.dtype)


def _fold(w, b, a, dtype=jnp.float32):
    g, m, n = w.shape
    r = b.shape[-1]
    return pl.pallas_call(
        _fold_body,
        grid=(g,),
        in_specs=[
            pl.BlockSpec((1, m, n), lambda i: (i, 0, 0)),
            pl.BlockSpec((1, m, r), lambda i: (i, 0, 0)),
            pl.BlockSpec((1, r, n), lambda i: (i, 0, 0)),
        ],
        out_specs=pl.BlockSpec((1, m, n), lambda i: (i, 0, 0)),
        out_shape=jax.ShapeDtypeStruct((g, m, n), dtype),
    )(w, b, a)


def _rope_body(c_ref, s_ref):
    t = c_ref.shape[0]
    pos = lax.broadcasted_iota(jnp.int32, (t, HEAD // 2), 0).astype(jnp.float32)
    j = lax.broadcasted_iota(jnp.int32, (t, HEAD // 2), 1).astype(jnp.float32)
    ang = pos * jnp.exp(j * (-2.0 * math.log(10000.0) / HEAD))
    c24 = jnp.cos(ang)
    s24 = jnp.sin(ang)
    jr = lax.broadcasted_iota(jnp.int32, (HEAD // 2, QKV_W), 0)
    lc = lax.broadcasted_iota(jnp.int32, (HEAD // 2, QKV_W), 1)
    rot = lc < (NQ + NKV)
    sel = (((lc % HEAD) // 2) == jr) & rot
    m = sel.astype(jnp.float32)
    sgn = jnp.where((lc % 2) == 0, -1.0, 1.0)
    vlane = jnp.where(rot, 0.0, 1.0)[0:1]
    c_ref[...] = jnp.dot(c24, m, preferred_element_type=jnp.float32) + vlane
    s_ref[...] = jnp.dot(s24, m * sgn, preferred_element_type=jnp.float32)


def _rope_tables(t):
    return pl.pallas_call(
        _rope_body,
        out_shape=(jax.ShapeDtypeStruct((t, QKV_W), jnp.float32),
                   jax.ShapeDtypeStruct((t, QKV_W), jnp.float32)),
    )()


def _qkv_body(x_ref, g_ref, b_ref, w_ref, c_ref, s_ref, o_ref):
    x = x_ref[...]
    m = jnp.mean(x, axis=-1, keepdims=True)
    v = jnp.mean((x - m) ** 2, axis=-1, keepdims=True)
    xn = (x - m) / jnp.sqrt(v + 1e-5) * g_ref[...] + b_ref[...]
    qkv = _dot_t(xn, w_ref[...])
    lane = lax.broadcasted_iota(jnp.int32, qkv.shape, 1)
    even = (lane % 2) == 0
    nl = qkv.shape[1]
    rot = jnp.where(even, pltpu.roll(qkv, nl - 1, 1), pltpu.roll(qkv, 1, 1))
    qkv = (qkv * c_ref[...] + rot * s_ref[...]).astype(jnp.bfloat16)
    for j in range(QKV_W // HEAD):
        o_ref[0, j] = qkv[:, j * HEAD:(j + 1) * HEAD]


def _qkv(x2d, g, b, w, c_tab, s_tab, bb, t, blk):
    bt = x2d.shape[0]
    tb = t // blk
    nh = QKV_W // HEAD
    return pl.pallas_call(
        _qkv_body,
        grid=(bt // blk,),
        in_specs=[
            pl.BlockSpec((blk, N_EMBD), lambda i: (i, 0)),
            pl.BlockSpec((1, N_EMBD), lambda i: (0, 0)),
            pl.BlockSpec((1, N_EMBD), lambda i: (0, 0)),
            pl.BlockSpec((QKV_W, N_EMBD), lambda i: (0, 0)),
            pl.BlockSpec((blk, QKV_W), lambda i: (i % tb, 0)),
            pl.BlockSpec((blk, QKV_W), lambda i: (i % tb, 0)),
        ],
        out_specs=pl.BlockSpec((1, nh, blk, HEAD),
                               lambda i: (i // tb, 0, i % tb, 0)),
        out_shape=jax.ShapeDtypeStruct((bb, nh, t, HEAD), jnp.bfloat16),
    )(x2d, g, b, w, c_tab, s_tab)


def _flash_body(q_ref, k_ref, v_ref, o_ref, m_ref, l_ref, acc_ref,
                *, blk_q, blk_k, nk):
    qi = pl.program_id(2)
    ki = pl.program_id(3)

    @pl.when(ki == 0)
    def _():
        m_ref[...] = jnp.full_like(m_ref, -1e30)
        l_ref[...] = jnp.zeros_like(l_ref)
        acc_ref[...] = jnp.zeros_like(acc_ref)

    @pl.when(ki * blk_k <= qi * blk_q + blk_q - 1)
    def _():
        q = q_ref[0, 0]
        s = lax.dot_general(q, k_ref[0, 0], (((1,), (1,)), ((), ())),
                            preferred_element_type=jnp.float32)
        s = s * (1.0 / math.sqrt(HEAD))
        qpos = qi * blk_q + lax.broadcasted_iota(jnp.int32, s.shape, 0)
        kpos = ki * blk_k + lax.broadcasted_iota(jnp.int32, s.shape, 1)
        s = jnp.where(kpos <= qpos, s, -1e30)
        m_prev = m_ref[:, 0:1]
        m_cur = jnp.max(s, axis=-1, keepdims=True)
        m_new = jnp.maximum(m_prev, m_cur)
        alpha = jnp.exp(m_prev - m_new)
        p = jnp.exp(s - m_new)
        l_ref[:, 0:1] = l_ref[:, 0:1] * alpha + jnp.sum(p, axis=-1,
                                                        keepdims=True)
        m_ref[:, 0:1] = m_new
        acc_ref[...] = acc_ref[...] * alpha + jnp.dot(
            p.astype(jnp.bfloat16), v_ref[0, 0],
            preferred_element_type=jnp.float32)

    @pl.when(ki == nk - 1)
    def _():
        o_ref[0, 0] = acc_ref[...] / l_ref[:, 0:1]


def _flash(qkvh, blk_q, blk_k):
    bb, _, t, d = qkvh.shape
    nq, nk = t // blk_q, t // blk_k
    rep = N_HEAD // N_KV
    kcol = NQ // HEAD
    vcol = (NQ + NKV) // HEAD
    return pl.pallas_call(
        functools.partial(_flash_body, blk_q=blk_q, blk_k=blk_k, nk=nk),
        grid=(bb, N_HEAD, nq, nk),
        in_specs=[
            pl.BlockSpec((1, 1, blk_q, d),
                         lambda b_, h_, qi, ki: (b_, h_, qi, 0)),
            pl.BlockSpec((1, 1, blk_k, d),
                         lambda b_, h_, qi, ki: (b_, kcol + h_ // rep, ki, 0)),
            pl.BlockSpec((1, 1, blk_k, d),
                         lambda b_, h_, qi, ki: (b_, vcol + h_ // rep, ki, 0)),
        ],
        out_specs=pl.BlockSpec((1, 1, blk_q, d),
                               lambda b_, h_, qi, ki: (b_, h_, qi, 0)),
        out_shape=jax.ShapeDtypeStruct((bb, N_HEAD, t, d), jnp.float32),
        scratch_shapes=[
            pltpu.VMEM((blk_q, 128), jnp.float32),
            pltpu.VMEM((blk_q, 128), jnp.float32),
            pltpu.VMEM((blk_q, d), jnp.float32),
        ],
    )(qkvh, qkvh, qkvh)


def _oproj_body(a_ref, x_ref, w_ref, g_ref, b_ref, h_ref, xn_ref):
    a = jnp.concatenate([a_ref[0, j] for j in range(N_HEAD)], axis=1)
    h = _dot_t(a, w_ref[...]) + x_ref[...]
    h_ref[...] = h
    m = jnp.mean(h, axis=-1, keepdims=True)
    v = jnp.mean((h - m) ** 2, axis=-1, keepdims=True)
    xn_ref[...] = (h - m) / jnp.sqrt(v + 1e-5) * g_ref[...] + b_ref[...]


def _oproj(attnh, x2d, wo, g, b, t, blk):
    bt = x2d.shape[0]
    tb = t // blk
    return pl.pallas_call(
        _oproj_body,
        grid=(bt // blk,),
        in_specs=[
            pl.BlockSpec((1, N_HEAD, blk, HEAD),
                         lambda i: (i // tb, 0, i % tb, 0)),
            pl.BlockSpec((blk, N_EMBD), lambda i: (i, 0)),
            pl.BlockSpec((N_EMBD, N_EMBD), lambda i: (0, 0)),
            pl.BlockSpec((1, N_EMBD), lambda i: (0, 0)),
            pl.BlockSpec((1, N_EMBD), lambda i: (0, 0)),
        ],
        out_specs=(pl.BlockSpec((blk, N_EMBD), lambda i: (i, 0)),
                   pl.BlockSpec((blk, N_EMBD), lambda i: (i, 0))),
        out_shape=(jax.ShapeDtypeStruct((bt, N_EMBD), jnp.float32),
                   jax.ShapeDtypeStruct((bt, N_EMBD), jnp.float32)),
    )(attnh, x2d, wo, g, b)


def _router_body(xn_ref, wr_ref, br_ref, dest_ref, gate_ref, aux_ref,
                 cnt_ref, psum_ref, lsum_ref, *, blk, nblk, cap, bt):
    i = pl.program_id(0)

    @pl.when(i == 0)
    def _():
        cnt_ref[...] = jnp.zeros_like(cnt_ref)
        psum_ref[...] = jnp.zeros_like(psum_ref)
        lsum_ref[...] = jnp.zeros_like(lsum_ref)

    xn = xn_ref[...]
    logits = _dot_t(xn, wr_ref[...]) + br_ref[...]
    mx = jnp.max(logits, axis=-1, keepdims=True)
    ex = jnp.exp(logits - mx)
    probs = ex / jnp.sum(ex, axis=-1, keepdims=True)
    top_v = jnp.max(probs, axis=-1, keepdims=True)
    lane = lax.broadcasted_iota(jnp.int32, probs.shape, 1)
    idx = jnp.min(jnp.where(probs >= top_v, lane, E), axis=-1,
                  keepdims=True)
    onehot = (lane == idx).astype(jnp.float32)
    row = lax.broadcasted_iota(jnp.int32, (blk, blk), 0)
    col = lax.broadcasted_iota(jnp.int32, (blk, blk), 1)
    tri = (row >= col).astype(jnp.float32)
    csum = jnp.dot(tri, onehot, preferred_element_type=jnp.float32)
    pos = csum - 1.0 + cnt_ref[0:1, 0:E]
    disp = onehot * (pos < cap).astype(jnp.float32)
    disp_tok = jnp.sum(disp, axis=-1, keepdims=True)
    pos_tok = jnp.sum(disp * pos, axis=-1, keepdims=True)
    rowid = i * blk + lax.broadcasted_iota(jnp.int32, (blk, 1), 0)
    dest_hit = idx * cap + pos_tok.astype(jnp.int32)
    trash = E * cap + rowid // (bt // SC_WORKERS)
    dest_ref[...] = jnp.where(disp_tok > 0.0, dest_hit, trash)
    gate_ref[...] = top_v * disp_tok
    cnt_ref[0:1, 0:E] = cnt_ref[0:1, 0:E] + jnp.sum(onehot, axis=0,
                                                    keepdims=True)
    psum_ref[0:1, 0:E] = psum_ref[0:1, 0:E] + jnp.sum(probs, axis=0,
                                                      keepdims=True)
    lsum_ref[0:1, 0:E] = lsum_ref[0:1, 0:E] + jnp.sum(disp, axis=0,
                                                      keepdims=True)

    @pl.when(i == nblk - 1)
    def _():
        prod = psum_ref[0:1, 0:E] * lsum_ref[0:1, 0:E]
        aux_ref[...] = jnp.sum(prod, axis=-1, keepdims=True) \
            * (float(E) / (float(bt) * float(bt)))


def _router(xn2, wr, br, cap, blk):
    bt = xn2.shape[0]
    nblk = bt // blk
    body = functools.partial(_router_body, blk=blk, nblk=nblk, cap=cap, bt=bt)
    return pl.pallas_call(
        body,
        grid=(nblk,),
        in_specs=[
            pl.BlockSpec((blk, N_EMBD), lambda i: (i, 0)),
            pl.BlockSpec((E, N_EMBD), lambda i: (0, 0)),
            pl.BlockSpec((1, E), lambda i: (0, 0)),
        ],
        out_specs=(pl.BlockSpec((blk, 1), lambda i: (i, 0)),
                   pl.BlockSpec((blk, 1), lambda i: (i, 0)),
                   pl.BlockSpec((1, 1), lambda i: (0, 0))),
        out_shape=(jax.ShapeDtypeStruct((bt, 1), jnp.int32),
                   jax.ShapeDtypeStruct((bt, 1), jnp.float32),
                   jax.ShapeDtypeStruct((1, 1), jnp.float32)),
        scratch_shapes=[
            pltpu.VMEM((1, 128), jnp.float32),
            pltpu.VMEM((1, 128), jnp.float32),
            pltpu.VMEM((1, 128), jnp.float32),
        ],
    )(xn2, wr, br)


def _sc_scatter(xn2, dest, nslot):
    bt, c = xn2.shape
    tpw = bt // SC_WORKERS

    @functools.partial(
        pl.kernel,
        out_type=jax.ShapeDtypeStruct((nslot, c), jnp.float32),
        mesh=_sc_mesh(),
        scratch_types=[
            pltpu.VMEM((tpw,), jnp.int32),
            pltpu.VMEM((tpw, c), jnp.float32),
            pltpu.SemaphoreType.DMA,
        ],
    )
    def k(xn_hbm, dest_hbm, buf_hbm, idx_v, rows_v, sem):
        wid = lax.axis_index("s") * SC_CORES + lax.axis_index("c")
        base = wid * tpw
        pltpu.sync_copy(dest_hbm.at[pl.ds(base, tpw)], idx_v)
        pltpu.sync_copy(xn_hbm.at[pl.ds(base, tpw)], rows_v)
        pltpu.async_copy(rows_v, buf_hbm.at[idx_v], sem).wait()

    return k(xn2, dest)


def _sc_gather(ybuf, dest, bt):
    c = ybuf.shape[1]
    tpw = bt // SC_WORKERS

    @functools.partial(
        pl.kernel,
        out_type=jax.ShapeDtypeStruct((bt, c), jnp.float32),
        mesh=_sc_mesh(),
        scratch_types=[
            pltpu.VMEM((tpw,), jnp.int32),
            pltpu.VMEM((tpw, c), jnp.float32),
            pltpu.SemaphoreType.DMA,
        ],
    )
    def k(ybuf_hbm, dest_hbm, out_hbm, idx_v, y_v, sem):
        wid = lax.axis_index("s") * SC_CORES + lax.axis_index("c")
        base = wid * tpw
        pltpu.sync_copy(dest_hbm.at[pl.ds(base, tpw)], idx_v)
        pltpu.async_copy(ybuf_hbm.at[idx_v], y_v, sem).wait()
        pltpu.sync_copy(y_v, out_hbm.at[pl.ds(base, tpw)])

    return k(ybuf, dest)


def _mlp_body(x_ref, w1_ref, w2_ref, y_ref):
    x = x_ref[...].astype(jnp.bfloat16)
    h = lax.dot_general(x, w1_ref[0], (((1,), (1,)), ((), ())),
                        preferred_element_type=jnp.float32)
    h = jnp.maximum(h, 0.0).astype(jnp.bfloat16)
    y_ref[...] = lax.dot_general(h, w2_ref[0], (((1,), (1,)), ((), ())),
                                 preferred_element_type=jnp.float32)


def _mlp(buf, w1, w2, cap, nslot, blk):
    nblk = cap // blk
    return pl.pallas_call(
        _mlp_body,
        grid=(E, nblk),
        in_specs=[
            pl.BlockSpec((blk, N_EMBD), lambda e, i: (e * nblk + i, 0)),
            pl.BlockSpec((1, HID, N_EMBD), lambda e, i: (e, 0, 0)),
            pl.BlockSpec((1, N_EMBD, HID), lambda e, i: (e, 0, 0)),
        ],
        out_specs=pl.BlockSpec((blk, N_EMBD), lambda e, i: (e * nblk + i, 0)),
        out_shape=jax.ShapeDtypeStruct((nslot, N_EMBD), jnp.float32),
    )(buf, w1, w2)


def _epi_body(h_ref, g_ref, y_ref, o_ref):
    g = g_ref[...]
    o_ref[...] = h_ref[...] + jnp.where(g > 0.0, g * y_ref[...], 0.0)


def _epilogue(h2d, gate, ygath, blk):
    bt = h2d.shape[0]
    return pl.pallas_call(
        _epi_body,
        grid=(bt // blk,),
        in_specs=[
            pl.BlockSpec((blk, N_EMBD), lambda i: (i, 0)),
            pl.BlockSpec((blk, 1), lambda i: (i, 0)),
            pl.BlockSpec((blk, N_EMBD), lambda i: (i, 0)),
        ],
        out_specs=pl.BlockSpec((blk, N_EMBD), lambda i: (i, 0)),
        out_shape=jax.ShapeDtypeStruct((bt, N_EMBD), jnp.float32),
    )(h2d, gate, ygath)


def kernel(x, adapter_id, params):
    p = params
    b, t, c = x.shape
    bt = b * t
    cap = int(math.ceil(CAP_F * bt / E))
    nslot = E * cap + SC_WORKERS
    aid = jnp.asarray(adapter_id).astype(jnp.int32)

    w_all = jnp.concatenate([p['Wq'], p['Wk'], p['Wv'], p['Wo']], axis=0)
    a_all = jnp.concatenate(
        [p['Aq'][aid], p['Ak'][aid], p['Av'][aid], p['Ao'][aid]], axis=0)
    b_bd = jnp.zeros((2 * NQ + 2 * NKV, 4 * R), jnp.float32)
    b_bd = b_bd.at[0:NQ, 0:R].set(p['Bq'][aid])
    b_bd = b_bd.at[NQ:NQ + NKV, R:2 * R].set(p['Bk'][aid])
    b_bd = b_bd.at[NQ + NKV:NQ + 2 * NKV, 2 * R:3 * R].set(p['Bv'][aid])
    b_bd = b_bd.at[NQ + 2 * NKV:, 3 * R:].set(p['Bo'][aid])
    qkvo = _fold(w_all[None], b_bd[None], a_all[None])[0]
    wqkv = qkvo[:QKV_W]
    wo = qkvo[QKV_W:]

    w1 = _fold(p['W1'], p['B1'][:, aid], p['A1'][:, aid],
               dtype=jnp.bfloat16)
    w2 = _fold(p['W2'], p['B2'][:, aid], p['A2'][:, aid],
               dtype=jnp.bfloat16)

    c_tab, s_tab = _rope_tables(t)
    x2d = x.reshape(bt, c)
    qkvh = _qkv(x2d, p['ln1_g'][None], p['ln1_b'][None], wqkv,
                c_tab, s_tab, b, t, blk=256)
    attnh = _flash(qkvh, blk_q=512, blk_k=512)
    h2d, xn2 = _oproj(attnh, x2d, wo, p['ln2_g'][None], p['ln2_b'][None],
                      t, blk=256)

    dest, gate, aux = _router(xn2, p['Wr'], p['br'][None], cap, blk=512)
    buf = _sc_scatter(xn2, dest.reshape(bt), nslot)
    ybuf = _mlp(buf, w1, w2, cap, nslot, blk=640)
    ygath = _sc_gather(ybuf, dest.reshape(bt), bt)
    out2d = _epilogue(h2d, gate, ygath, blk=512)
    return out2d.reshape(b, t, c), aux.reshape(())


# speedup vs baseline: 1.5735x; 1.1320x over previous
import functools
import math

import jax
import jax.numpy as jnp
from jax import lax
from jax.experimental import pallas as pl
from jax.experimental.pallas import tpu as pltpu
from jax.experimental.pallas import tpu_sc as plsc

N_EMBD = 384
N_HEAD = 8
N_KV = 2
HEAD = N_EMBD // N_HEAD
R = 4
E = 4
CAP_F = 1.25
LORA_SCALE = 1.0 / R
HID = 4 * N_EMBD
NQ = N_HEAD * HEAD
NKV = N_KV * HEAD
QKV_W = NQ + 2 * NKV

SC_CORES = 2
SC_SUBCORES = 16
SC_WORKERS = SC_CORES * SC_SUBCORES


def _sc_mesh():
    return plsc.VectorSubcoreMesh(
        core_axis_name="c", subcore_axis_name="s",
        num_cores=SC_CORES, num_subcores=SC_SUBCORES)


def _dot_t(x, w):
    return lax.dot_general(x, w, (((1,), (1,)), ((), ())),
                           preferred_element_type=jnp.float32)


def _fold_body(w_ref, b_ref, a_ref, o_ref):
    eff = w_ref[0] + LORA_SCALE * jnp.dot(
        b_ref[0], a_ref[0], preferred_element_type=jnp.float32)
    o_ref[0] = eff.astype(o_ref.dtype)


def _fold(w, b, a, dtype=jnp.float32):
    g, m, n = w.shape
    r = b.shape[-1]
    return pl.pallas_call(
        _fold_body,
        grid=(g,),
        in_specs=[
            pl.BlockSpec((1, m, n), lambda i: (i, 0, 0)),
            pl.BlockSpec((1, m, r), lambda i: (i, 0, 0)),
            pl.BlockSpec((1, r, n), lambda i: (i, 0, 0)),
        ],
        out_specs=pl.BlockSpec((1, m, n), lambda i: (i, 0, 0)),
        out_shape=jax.ShapeDtypeStruct((g, m, n), dtype),
    )(w, b, a)


def _rope_body(c_ref, s_ref):
    t = c_ref.shape[0]
    pos = lax.broadcasted_iota(jnp.int32, (t, HEAD // 2), 0).astype(jnp.float32)
    j = lax.broadcasted_iota(jnp.int32, (t, HEAD // 2), 1).astype(jnp.float32)
    ang = pos * jnp.exp(j * (-2.0 * math.log(10000.0) / HEAD))
    c24 = jnp.cos(ang)
    s24 = jnp.sin(ang)
    jr = lax.broadcasted_iota(jnp.int32, (HEAD // 2, QKV_W), 0)
    lc = lax.broadcasted_iota(jnp.int32, (HEAD // 2, QKV_W), 1)
    rot = lc < (NQ + NKV)
    sel = (((lc % HEAD) // 2) == jr) & rot
    m = sel.astype(jnp.float32)
    sgn = jnp.where((lc % 2) == 0, -1.0, 1.0)
    vlane = jnp.where(rot, 0.0, 1.0)[0:1]
    qscale = jnp.where(lc < NQ, 1.0 / math.sqrt(HEAD), 1.0)[0:1]
    c_full = jnp.dot(c24, m, preferred_element_type=jnp.float32) + vlane
    s_full = jnp.dot(s24, m * sgn, preferred_element_type=jnp.float32)
    c_ref[...] = c_full * qscale
    s_ref[...] = s_full * qscale


def _rope_tables(t):
    return pl.pallas_call(
        _rope_body,
        out_shape=(jax.ShapeDtypeStruct((t, QKV_W), jnp.float32),
                   jax.ShapeDtypeStruct((t, QKV_W), jnp.float32)),
    )()


def _qkv_body(x_ref, g_ref, b_ref, w_ref, c_ref, s_ref, o_ref):
    x = x_ref[...]
    m = jnp.mean(x, axis=-1, keepdims=True)
    v = jnp.mean((x - m) ** 2, axis=-1, keepdims=True)
    xn = (x - m) / jnp.sqrt(v + 1e-5) * g_ref[...] + b_ref[...]
    qkv = _dot_t(xn, w_ref[...])
    lane = lax.broadcasted_iota(jnp.int32, qkv.shape, 1)
    even = (lane % 2) == 0
    nl = qkv.shape[1]
    rot = jnp.where(even, pltpu.roll(qkv, nl - 1, 1), pltpu.roll(qkv, 1, 1))
    qkv = (qkv * c_ref[...] + rot * s_ref[...]).astype(jnp.bfloat16)
    for j in range(QKV_W // HEAD):
        o_ref[0, j] = qkv[:, j * HEAD:(j + 1) * HEAD]


def _qkv(x2d, g, b, w, c_tab, s_tab, bb, t, blk):
    bt = x2d.shape[0]
    tb = t // blk
    nh = QKV_W // HEAD
    return pl.pallas_call(
        _qkv_body,
        grid=(bt // blk,),
        in_specs=[
            pl.BlockSpec((blk, N_EMBD), lambda i: (i, 0)),
            pl.BlockSpec((1, N_EMBD), lambda i: (0, 0)),
            pl.BlockSpec((1, N_EMBD), lambda i: (0, 0)),
            pl.BlockSpec((QKV_W, N_EMBD), lambda i: (0, 0)),
            pl.BlockSpec((blk, QKV_W), lambda i: (i % tb, 0)),
            pl.BlockSpec((blk, QKV_W), lambda i: (i % tb, 0)),
        ],
        out_specs=pl.BlockSpec((1, nh, blk, HEAD),
                               lambda i: (i // tb, 0, i % tb, 0)),
        out_shape=jax.ShapeDtypeStruct((bb, nh, t, HEAD), jnp.bfloat16),
    )(x2d, g, b, w, c_tab, s_tab)


def _tri_qk(pid, nq):
    qi = jnp.zeros((), jnp.int32)
    for q in range(1, nq):
        qi = qi + (pid >= (q * (q + 1)) // 2).astype(jnp.int32)
    ki = pid - qi * (qi + 1) // 2
    return qi, ki


def _flash_body(q_ref, k_ref, v_ref, o_ref, m_ref, l_ref, acc_ref,
                *, blk, nq):
    pid = pl.program_id(2)
    qi, ki = _tri_qk(pid, nq)

    @pl.when(ki == 0)
    def _():
        m_ref[...] = jnp.full_like(m_ref, -1e30)
        l_ref[...] = jnp.zeros_like(l_ref)
        acc_ref[...] = jnp.zeros_like(acc_ref)

    def update(s):
        m_prev = m_ref[:, 0:1]
        m_cur = jnp.max(s, axis=-1, keepdims=True)
        m_new = jnp.maximum(m_prev, m_cur)
        alpha = jnp.exp(m_prev - m_new)
        p = jnp.exp(s - m_new)
        l_ref[:, 0:1] = l_ref[:, 0:1] * alpha + jnp.sum(p, axis=-1,
                                                        keepdims=True)
        m_ref[:, 0:1] = m_new
        acc_ref[...] = acc_ref[...] * alpha + jnp.dot(
            p.astype(jnp.bfloat16), v_ref[0, 0],
            preferred_element_type=jnp.float32)

    @pl.when(ki < qi)
    def _():
        s = lax.dot_general(q_ref[0, 0], k_ref[0, 0],
                            (((1,), (1,)), ((), ())),
                            preferred_element_type=jnp.float32)
        update(s)

    @pl.when(ki == qi)
    def _():
        s = lax.dot_general(q_ref[0, 0], k_ref[0, 0],
                            (((1,), (1,)), ((), ())),
                            preferred_element_type=jnp.float32)
        r = lax.broadcasted_iota(jnp.int32, s.shape, 0)
        c = lax.broadcasted_iota(jnp.int32, s.shape, 1)
        update(jnp.where(c <= r, s, -1e30))
        o_ref[0, 0] = acc_ref[...] / l_ref[:, 0:1]


def _flash(qkvh, blk):
    bb, _, t, d = qkvh.shape
    nq = t // blk
    npair = nq * (nq + 1) // 2
    rep = N_HEAD // N_KV
    kcol = NQ // HEAD
    vcol = (NQ + NKV) // HEAD

    def qmap(b_, h_, p_):
        qi, _ = _tri_qk(p_, nq)
        return (b_, h_, qi, 0)

    def kmap(b_, h_, p_):
        _, ki = _tri_qk(p_, nq)
        return (b_, kcol + h_ // rep, ki, 0)

    def vmap_(b_, h_, p_):
        _, ki = _tri_qk(p_, nq)
        return (b_, vcol + h_ // rep, ki, 0)

    return pl.pallas_call(
        functools.partial(_flash_body, blk=blk, nq=nq),
        grid=(bb, N_HEAD, npair),
        in_specs=[
            pl.BlockSpec((1, 1, blk, d), qmap),
            pl.BlockSpec((1, 1, blk, d), kmap),
            pl.BlockSpec((1, 1, blk, d), vmap_),
        ],
        out_specs=pl.BlockSpec((1, 1, blk, d), qmap),
        out_shape=jax.ShapeDtypeStruct((bb, N_HEAD, t, d), jnp.float32),
        scratch_shapes=[
            pltpu.VMEM((blk, 128), jnp.float32),
            pltpu.VMEM((blk, 128), jnp.float32),
            pltpu.VMEM((blk, d), jnp.float32),
        ],
    )(qkvh, qkvh, qkvh)


def _oproj_body(a_ref, x_ref, w_ref, g_ref, b_ref, h_ref, xn_ref):
    a = jnp.concatenate([a_ref[0, j] for j in range(N_HEAD)], axis=1)
    h = _dot_t(a, w_ref[...]) + x_ref[...]
    h_ref[...] = h
    m = jnp.mean(h, axis=-1, keepdims=True)
    v = jnp.mean((h - m) ** 2, axis=-1, keepdims=True)
    xn_ref[...] = (h - m) / jnp.sqrt(v + 1e-5) * g_ref[...] + b_ref[...]


def _oproj(attnh, x2d, wo, g, b, t, blk):
    bt = x2d.shape[0]
    tb = t // blk
    return pl.pallas_call(
        _oproj_body,
        grid=(bt // blk,),
        in_specs=[
            pl.BlockSpec((1, N_HEAD, blk, HEAD),
                         lambda i: (i // tb, 0, i % tb, 0)),
            pl.BlockSpec((blk, N_EMBD), lambda i: (i, 0)),
            pl.BlockSpec((N_EMBD, N_EMBD), lambda i: (0, 0)),
            pl.BlockSpec((1, N_EMBD), lambda i: (0, 0)),
            pl.BlockSpec((1, N_EMBD), lambda i: (0, 0)),
        ],
        out_specs=(pl.BlockSpec((blk, N_EMBD), lambda i: (i, 0)),
                   pl.BlockSpec((blk, N_EMBD), lambda i: (i, 0))),
        out_shape=(jax.ShapeDtypeStruct((bt, N_EMBD), jnp.float32),
                   jax.ShapeDtypeStruct((bt, N_EMBD), jnp.float32)),
    )(attnh, x2d, wo, g, b)


def _router_body(xn_ref, wr_ref, br_ref, dest_ref, gate_ref, aux_ref,
                 cnt_ref, psum_ref, lsum_ref, *, blk, nblk, cap, bt):
    i = pl.program_id(0)

    @pl.when(i == 0)
    def _():
        cnt_ref[...] = jnp.zeros_like(cnt_ref)
        psum_ref[...] = jnp.zeros_like(psum_ref)
        lsum_ref[...] = jnp.zeros_like(lsum_ref)

    xn = xn_ref[...]
    logits = _dot_t(xn, wr_ref[...]) + br_ref[...]
    mx = jnp.max(logits, axis=-1, keepdims=True)
    ex = jnp.exp(logits - mx)
    probs = ex / jnp.sum(ex, axis=-1, keepdims=True)
    top_v = jnp.max(probs, axis=-1, keepdims=True)
    lane = lax.broadcasted_iota(jnp.int32, probs.shape, 1)
    idx = jnp.min(jnp.where(probs >= top_v, lane, E), axis=-1,
                  keepdims=True)
    onehot = (lane == idx).astype(jnp.float32)
    row = lax.broadcasted_iota(jnp.int32, (blk, blk), 0)
    col = lax.broadcasted_iota(jnp.int32, (blk, blk), 1)
    tri = (row >= col).astype(jnp.float32)
    csum = jnp.dot(tri, onehot, preferred_element_type=jnp.float32)
    pos = csum - 1.0 + cnt_ref[0:1, 0:E]
    disp = onehot * (pos < cap).astype(jnp.float32)
    disp_tok = jnp.sum(disp, axis=-1, keepdims=True)
    pos_tok = jnp.sum(disp * pos, axis=-1, keepdims=True)
    rowid = i * blk + lax.broadcasted_iota(jnp.int32, (blk, 1), 0)
    dest_hit = idx * cap + pos_tok.astype(jnp.int32)
    trash = E * cap + rowid // (bt // SC_WORKERS)
    dest_ref[...] = jnp.where(disp_tok > 0.0, dest_hit, trash)
    gate_ref[...] = top_v * disp_tok
    cnt_ref[0:1, 0:E] = cnt_ref[0:1, 0:E] + jnp.sum(onehot, axis=0,
                                                    keepdims=True)
    psum_ref[0:1, 0:E] = psum_ref[0:1, 0:E] + jnp.sum(probs, axis=0,
                                                      keepdims=True)
    lsum_ref[0:1, 0:E] = lsum_ref[0:1, 0:E] + jnp.sum(disp, axis=0,
                                                      keepdims=True)

    @pl.when(i == nblk - 1)
    def _():
        prod = psum_ref[0:1, 0:E] * lsum_ref[0:1, 0:E]
        aux_ref[...] = jnp.sum(prod, axis=-1, keepdims=True) \
            * (float(E) / (float(bt) * float(bt)))


def _router(xn2, wr, br, cap, blk):
    bt = xn2.shape[0]
    nblk = bt // blk
    body = functools.partial(_router_body, blk=blk, nblk=nblk, cap=cap, bt=bt)
    return pl.pallas_call(
        body,
        grid=(nblk,),
        in_specs=[
            pl.BlockSpec((blk, N_EMBD), lambda i: (i, 0)),
            pl.BlockSpec((E, N_EMBD), lambda i: (0, 0)),
            pl.BlockSpec((1, E), lambda i: (0, 0)),
        ],
        out_specs=(pl.BlockSpec((blk, 1), lambda i: (i, 0)),
                   pl.BlockSpec((blk, 1), lambda i: (i, 0)),
                   pl.BlockSpec((1, 1), lambda i: (0, 0))),
        out_shape=(jax.ShapeDtypeStruct((bt, 1), jnp.int32),
                   jax.ShapeDtypeStruct((bt, 1), jnp.float32),
                   jax.ShapeDtypeStruct((1, 1), jnp.float32)),
        scratch_shapes=[
            pltpu.VMEM((1, 128), jnp.float32),
            pltpu.VMEM((1, 128), jnp.float32),
            pltpu.VMEM((1, 128), jnp.float32),
        ],
    )(xn2, wr, br)


def _sc_scatter(xn2, dest, nslot):
    bt, c = xn2.shape
    tpw = bt // SC_WORKERS

    @functools.partial(
        pl.kernel,
        out_type=jax.ShapeDtypeStruct((nslot, c), jnp.float32),
        mesh=_sc_mesh(),
        scratch_types=[
            pltpu.VMEM((tpw,), jnp.int32),
            pltpu.VMEM((tpw, c), jnp.float32),
            pltpu.SemaphoreType.DMA,
        ],
    )
    def k(xn_hbm, dest_hbm, buf_hbm, idx_v, rows_v, sem):
        wid = lax.axis_index("s") * SC_CORES + lax.axis_index("c")
        base = wid * tpw
        pltpu.sync_copy(dest_hbm.at[pl.ds(base, tpw)], idx_v)
        pltpu.sync_copy(xn_hbm.at[pl.ds(base, tpw)], rows_v)
        pltpu.async_copy(rows_v, buf_hbm.at[idx_v], sem).wait()

    return k(xn2, dest)


def _sc_gather(ybuf, dest, bt):
    c = ybuf.shape[1]
    tpw = bt // SC_WORKERS

    @functools.partial(
        pl.kernel,
        out_type=jax.ShapeDtypeStruct((bt, c), jnp.float32),
        mesh=_sc_mesh(),
        scratch_types=[
            pltpu.VMEM((tpw,), jnp.int32),
            pltpu.VMEM((tpw, c), jnp.float32),
            pltpu.SemaphoreType.DMA,
        ],
    )
    def k(ybuf_hbm, dest_hbm, out_hbm, idx_v, y_v, sem):
        wid = lax.axis_index("s") * SC_CORES + lax.axis_index("c")
        base = wid * tpw
        pltpu.sync_copy(dest_hbm.at[pl.ds(base, tpw)], idx_v)
        pltpu.async_copy(ybuf_hbm.at[idx_v], y_v, sem).wait()
        pltpu.sync_copy(y_v, out_hbm.at[pl.ds(base, tpw)])

    return k(ybuf, dest)


def _mlp_body(x_ref, w1_ref, w2_ref, y_ref):
    x = x_ref[...].astype(jnp.bfloat16)
    h = lax.dot_general(x, w1_ref[0], (((1,), (1,)), ((), ())),
                        preferred_element_type=jnp.float32)
    h = jnp.maximum(h, 0.0).astype(jnp.bfloat16)
    y_ref[...] = lax.dot_general(h, w2_ref[0], (((1,), (1,)), ((), ())),
                                 preferred_element_type=jnp.float32)


def _mlp(buf, w1, w2, cap, nslot, blk):
    nblk = cap // blk
    return pl.pallas_call(
        _mlp_body,
        grid=(E, nblk),
        in_specs=[
            pl.BlockSpec((blk, N_EMBD), lambda e, i: (e * nblk + i, 0)),
            pl.BlockSpec((1, HID, N_EMBD), lambda e, i: (e, 0, 0)),
            pl.BlockSpec((1, N_EMBD, HID), lambda e, i: (e, 0, 0)),
        ],
        out_specs=pl.BlockSpec((blk, N_EMBD), lambda e, i: (e * nblk + i, 0)),
        out_shape=jax.ShapeDtypeStruct((nslot, N_EMBD), jnp.float32),
    )(buf, w1, w2)


def _epi_body(h_ref, g_ref, y_ref, o_ref):
    g = g_ref[...]
    o_ref[...] = h_ref[...] + jnp.where(g > 0.0, g * y_ref[...], 0.0)


def _epilogue(h2d, gate, ygath, blk):
    bt = h2d.shape[0]
    return pl.pallas_call(
        _epi_body,
        grid=(bt // blk,),
        in_specs=[
            pl.BlockSpec((blk, N_EMBD), lambda i: (i, 0)),
            pl.BlockSpec((blk, 1), lambda i: (i, 0)),
            pl.BlockSpec((blk, N_EMBD), lambda i: (i, 0)),
        ],
        out_specs=pl.BlockSpec((blk, N_EMBD), lambda i: (i, 0)),
        out_shape=jax.ShapeDtypeStruct((bt, N_EMBD), jnp.float32),
    )(h2d, gate, ygath)


def kernel(x, adapter_id, params):
    p = params
    b, t, c = x.shape
    bt = b * t
    cap = int(math.ceil(CAP_F * bt / E))
    nslot = E * cap + SC_WORKERS
    aid = jnp.asarray(adapter_id).astype(jnp.int32)

    w_all = jnp.concatenate([p['Wq'], p['Wk'], p['Wv'], p['Wo']], axis=0)
    a_all = jnp.concatenate(
        [p['Aq'][aid], p['Ak'][aid], p['Av'][aid], p['Ao'][aid]], axis=0)
    b_bd = jnp.zeros((2 * NQ + 2 * NKV, 4 * R), jnp.float32)
    b_bd = b_bd.at[0:NQ, 0:R].set(p['Bq'][aid])
    b_bd = b_bd.at[NQ:NQ + NKV, R:2 * R].set(p['Bk'][aid])
    b_bd = b_bd.at[NQ + NKV:NQ + 2 * NKV, 2 * R:3 * R].set(p['Bv'][aid])
    b_bd = b_bd.at[NQ + 2 * NKV:, 3 * R:].set(p['Bo'][aid])
    qkvo = _fold(w_all[None], b_bd[None], a_all[None])[0]
    wqkv = qkvo[:QKV_W]
    wo = qkvo[QKV_W:]

    w1 = _fold(p['W1'], p['B1'][:, aid], p['A1'][:, aid],
               dtype=jnp.bfloat16)
    w2 = _fold(p['W2'], p['B2'][:, aid], p['A2'][:, aid],
               dtype=jnp.bfloat16)

    c_tab, s_tab = _rope_tables(t)
    x2d = x.reshape(bt, c)
    qkvh = _qkv(x2d, p['ln1_g'][None], p['ln1_b'][None], wqkv,
                c_tab, s_tab, b, t, blk=512)
    attnh = _flash(qkvh, blk=512)
    h2d, xn2 = _oproj(attnh, x2d, wo, p['ln2_g'][None], p['ln2_b'][None],
                      t, blk=512)

    dest, gate, aux = _router(xn2, p['Wr'], p['br'][None], cap, blk=512)
    buf = _sc_scatter(xn2, dest.reshape(bt), nslot)
    ybuf = _mlp(buf, w1, w2, cap, nslot, blk=1280)
    ygath = _sc_gather(ybuf, dest.reshape(bt), bt)
    out2d = _epilogue(h2d, gate, ygath, blk=512)
    return out2d.reshape(b, t, c), aux.reshape(())


# speedup vs baseline: 1.6190x; 1.0289x over previous
import functools
import math

import jax
import jax.numpy as jnp
from jax import lax
from jax.experimental import pallas as pl
from jax.experimental.pallas import tpu as pltpu
from jax.experimental.pallas import tpu_sc as plsc

N_EMBD = 384
N_HEAD = 8
N_KV = 2
HEAD = N_EMBD // N_HEAD
R = 4
E = 4
CAP_F = 1.25
LORA_SCALE = 1.0 / R
HID = 4 * N_EMBD
NQ = N_HEAD * HEAD
NKV = N_KV * HEAD
QKV_W = NQ + 2 * NKV

SC_CORES = 2
SC_SUBCORES = 16
SC_WORKERS = SC_CORES * SC_SUBCORES


def _sc_mesh():
    return plsc.VectorSubcoreMesh(
        core_axis_name="c", subcore_axis_name="s",
        num_cores=SC_CORES, num_subcores=SC_SUBCORES)


def _dot_t(x, w):
    return lax.dot_general(x, w, (((1,), (1,)), ((), ())),
                           preferred_element_type=jnp.float32)


def _fold_body(w_ref, b_ref, a_ref, o_ref):
    eff = w_ref[0] + LORA_SCALE * jnp.dot(
        b_ref[0], a_ref[0], preferred_element_type=jnp.float32)
    o_ref[0] = eff.astype(o_ref.dtype)


def _fold(w, b, a, dtype=jnp.float32):
    g, m, n = w.shape
    r = b.shape[-1]
    return pl.pallas_call(
        _fold_body,
        grid=(g,),
        in_specs=[
            pl.BlockSpec((1, m, n), lambda i: (i, 0, 0)),
            pl.BlockSpec((1, m, r), lambda i: (i, 0, 0)),
            pl.BlockSpec((1, r, n), lambda i: (i, 0, 0)),
        ],
        out_specs=pl.BlockSpec((1, m, n), lambda i: (i, 0, 0)),
        out_shape=jax.ShapeDtypeStruct((g, m, n), dtype),
    )(w, b, a)


def _rope_body(c_ref, s_ref):
    t = c_ref.shape[0]
    pos = lax.broadcasted_iota(jnp.int32, (t, HEAD // 2), 0).astype(jnp.float32)
    j = lax.broadcasted_iota(jnp.int32, (t, HEAD // 2), 1).astype(jnp.float32)
    ang = pos * jnp.exp(j * (-2.0 * math.log(10000.0) / HEAD))
    c24 = jnp.cos(ang)
    s24 = jnp.sin(ang)
    jr = lax.broadcasted_iota(jnp.int32, (HEAD // 2, QKV_W), 0)
    lc = lax.broadcasted_iota(jnp.int32, (HEAD // 2, QKV_W), 1)
    rot = lc < (NQ + NKV)
    sel = (((lc % HEAD) // 2) == jr) & rot
    m = sel.astype(jnp.float32)
    sgn = jnp.where((lc % 2) == 0, -1.0, 1.0)
    vlane = jnp.where(rot, 0.0, 1.0)[0:1]
    qscale = jnp.where(lc < NQ, math.log2(math.e) / math.sqrt(HEAD),
                       1.0)[0:1]
    c_full = jnp.dot(c24, m, preferred_element_type=jnp.float32) + vlane
    s_full = jnp.dot(s24, m * sgn, preferred_element_type=jnp.float32)
    c_ref[...] = c_full * qscale
    s_ref[...] = s_full * qscale


def _rope_tables(t):
    return pl.pallas_call(
        _rope_body,
        out_shape=(jax.ShapeDtypeStruct((t, QKV_W), jnp.float32),
                   jax.ShapeDtypeStruct((t, QKV_W), jnp.float32)),
    )()


def _qkv_body(x_ref, g_ref, b_ref, w_ref, c_ref, s_ref, o_ref):
    x = x_ref[...]
    m = jnp.mean(x, axis=-1, keepdims=True)
    v = jnp.mean((x - m) ** 2, axis=-1, keepdims=True)
    xn = (x - m) / jnp.sqrt(v + 1e-5) * g_ref[...] + b_ref[...]
    qkv = lax.dot_general(
        xn.astype(jnp.bfloat16), w_ref[...].astype(jnp.bfloat16),
        (((1,), (1,)), ((), ())), preferred_element_type=jnp.float32)
    lane = lax.broadcasted_iota(jnp.int32, qkv.shape, 1)
    even = (lane % 2) == 0
    nl = qkv.shape[1]
    rot = jnp.where(even, pltpu.roll(qkv, nl - 1, 1), pltpu.roll(qkv, 1, 1))
    qkv = (qkv * c_ref[...] + rot * s_ref[...]).astype(jnp.bfloat16)
    for j in range(QKV_W // HEAD):
        o_ref[0, j] = qkv[:, j * HEAD:(j + 1) * HEAD]


def _qkv(x2d, g, b, w, c_tab, s_tab, bb, t, blk):
    bt = x2d.shape[0]
    tb = t // blk
    nh = QKV_W // HEAD
    return pl.pallas_call(
        _qkv_body,
        grid=(bt // blk,),
        in_specs=[
            pl.BlockSpec((blk, N_EMBD), lambda i: (i, 0)),
            pl.BlockSpec((1, N_EMBD), lambda i: (0, 0)),
            pl.BlockSpec((1, N_EMBD), lambda i: (0, 0)),
            pl.BlockSpec((QKV_W, N_EMBD), lambda i: (0, 0)),
            pl.BlockSpec((blk, QKV_W), lambda i: (i % tb, 0)),
            pl.BlockSpec((blk, QKV_W), lambda i: (i % tb, 0)),
        ],
        out_specs=pl.BlockSpec((1, nh, blk, HEAD),
                               lambda i: (i // tb, 0, i % tb, 0)),
        out_shape=jax.ShapeDtypeStruct((bb, nh, t, HEAD), jnp.bfloat16),
    )(x2d, g, b, w, c_tab, s_tab)


def _tri_qk(pid, nq):
    qi = jnp.zeros((), jnp.int32)
    for q in range(1, nq):
        qi = qi + (pid >= (q * (q + 1)) // 2).astype(jnp.int32)
    ki = pid - qi * (qi + 1) // 2
    return qi, ki


def _flash_body(q_ref, k_ref, v_ref, o_ref, m_ref, l_ref, acc_ref,
                *, blk, nq):
    pid = pl.program_id(2)
    qi, ki = _tri_qk(pid, nq)

    @pl.when(ki == 0)
    def _():
        m_ref[...] = jnp.full_like(m_ref, -1e30)
        l_ref[...] = jnp.zeros_like(l_ref)
        acc_ref[...] = jnp.zeros_like(acc_ref)

    def update(s):
        m_prev = m_ref[:, 0:1]
        m_cur = jnp.max(s, axis=-1, keepdims=True)
        m_new = jnp.maximum(m_prev, m_cur)
        alpha = jnp.exp2(m_prev - m_new)
        p = jnp.exp2(s - m_new)
        l_ref[:, 0:1] = l_ref[:, 0:1] * alpha + jnp.sum(p, axis=-1,
                                                        keepdims=True)
        m_ref[:, 0:1] = m_new
        acc_ref[...] = acc_ref[...] * alpha + jnp.dot(
            p.astype(jnp.bfloat16), v_ref[0, 0],
            preferred_element_type=jnp.float32)

    @pl.when(ki < qi)
    def _():
        s = lax.dot_general(q_ref[0, 0], k_ref[0, 0],
                            (((1,), (1,)), ((), ())),
                            preferred_element_type=jnp.float32)
        update(s)

    @pl.when(ki == qi)
    def _():
        s = lax.dot_general(q_ref[0, 0], k_ref[0, 0],
                            (((1,), (1,)), ((), ())),
                            preferred_element_type=jnp.float32)
        r = lax.broadcasted_iota(jnp.int32, s.shape, 0)
        c = lax.broadcasted_iota(jnp.int32, s.shape, 1)
        update(jnp.where(c <= r, s, -1e30))
        o_ref[0, 0] = (acc_ref[...] / l_ref[:, 0:1]).astype(jnp.bfloat16)


def _flash(qkvh, blk):
    bb, _, t, d = qkvh.shape
    nq = t // blk
    npair = nq * (nq + 1) // 2
    rep = N_HEAD // N_KV
    kcol = NQ // HEAD
    vcol = (NQ + NKV) // HEAD

    def qmap(b_, h_, p_):
        qi, _ = _tri_qk(p_, nq)
        return (b_, h_, qi, 0)

    def kmap(b_, h_, p_):
        _, ki = _tri_qk(p_, nq)
        return (b_, kcol + h_ // rep, ki, 0)

    def vmap_(b_, h_, p_):
        _, ki = _tri_qk(p_, nq)
        return (b_, vcol + h_ // rep, ki, 0)

    return pl.pallas_call(
        functools.partial(_flash_body, blk=blk, nq=nq),
        grid=(bb, N_HEAD, npair),
        in_specs=[
            pl.BlockSpec((1, 1, blk, d), qmap),
            pl.BlockSpec((1, 1, blk, d), kmap),
            pl.BlockSpec((1, 1, blk, d), vmap_),
        ],
        out_specs=pl.BlockSpec((1, 1, blk, d), qmap),
        out_shape=jax.ShapeDtypeStruct((bb, N_HEAD, t, d), jnp.bfloat16),
        scratch_shapes=[
            pltpu.VMEM((blk, 128), jnp.float32),
            pltpu.VMEM((blk, 128), jnp.float32),
            pltpu.VMEM((blk, d), jnp.float32),
        ],
    )(qkvh, qkvh, qkvh)


def _oproj_router_body(a_ref, x_ref, w_ref, g_ref, b_ref, wr_ref, br_ref,
                       h_ref, xn_ref, dest_ref, gate_ref, aux_ref,
                       cnt_ref, psum_ref, lsum_ref, *, blk, nblk, cap, bt):
    i = pl.program_id(0)

    @pl.when(i == 0)
    def _():
        cnt_ref[...] = jnp.zeros_like(cnt_ref)
        psum_ref[...] = jnp.zeros_like(psum_ref)
        lsum_ref[...] = jnp.zeros_like(lsum_ref)

    a = jnp.concatenate([a_ref[0, j] for j in range(N_HEAD)], axis=1)
    h = lax.dot_general(a, w_ref[...].astype(jnp.bfloat16),
                        (((1,), (1,)), ((), ())),
                        preferred_element_type=jnp.float32) + x_ref[...]
    h_ref[...] = h
    m = jnp.mean(h, axis=-1, keepdims=True)
    v = jnp.mean((h - m) ** 2, axis=-1, keepdims=True)
    xn = (h - m) / jnp.sqrt(v + 1e-5) * g_ref[...] + b_ref[...]
    xn_ref[...] = xn
    logits = _dot_t(xn, wr_ref[...]) + br_ref[...]
    mx = jnp.max(logits, axis=-1, keepdims=True)
    ex = jnp.exp(logits - mx)
    probs = ex / jnp.sum(ex, axis=-1, keepdims=True)
    top_v = jnp.max(probs, axis=-1, keepdims=True)
    lane = lax.broadcasted_iota(jnp.int32, probs.shape, 1)
    idx = jnp.min(jnp.where(probs >= top_v, lane, E), axis=-1,
                  keepdims=True)
    onehot = (lane == idx).astype(jnp.float32)
    row = lax.broadcasted_iota(jnp.int32, (blk, blk), 0)
    col = lax.broadcasted_iota(jnp.int32, (blk, blk), 1)
    tri = (row >= col).astype(jnp.float32)
    csum = jnp.dot(tri, onehot, preferred_element_type=jnp.float32)
    pos = csum - 1.0 + cnt_ref[0:1, 0:E]
    disp = onehot * (pos < cap).astype(jnp.float32)
    disp_tok = jnp.sum(disp, axis=-1, keepdims=True)
    pos_tok = jnp.sum(disp * pos, axis=-1, keepdims=True)
    rowid = i * blk + lax.broadcasted_iota(jnp.int32, (blk, 1), 0)
    dest_hit = idx * cap + pos_tok.astype(jnp.int32)
    trash = E * cap + rowid // (bt // SC_WORKERS)
    dest_ref[...] = jnp.where(disp_tok > 0.0, dest_hit, trash)
    gate_ref[...] = top_v * disp_tok
    cnt_ref[0:1, 0:E] = cnt_ref[0:1, 0:E] + jnp.sum(onehot, axis=0,
                                                    keepdims=True)
    psum_ref[0:1, 0:E] = psum_ref[0:1, 0:E] + jnp.sum(probs, axis=0,
                                                      keepdims=True)
    lsum_ref[0:1, 0:E] = lsum_ref[0:1, 0:E] + jnp.sum(disp, axis=0,
                                                      keepdims=True)

    @pl.when(i == nblk - 1)
    def _():
        prod = psum_ref[0:1, 0:E] * lsum_ref[0:1, 0:E]
        aux_ref[...] = jnp.sum(prod, axis=-1, keepdims=True) \
            * (float(E) / (float(bt) * float(bt)))


def _oproj_router(attnh, x2d, wo, g, b, wr, br, cap, t, blk):
    bt = x2d.shape[0]
    tb = t // blk
    nblk = bt // blk
    body = functools.partial(_oproj_router_body, blk=blk, nblk=nblk,
                             cap=cap, bt=bt)
    return pl.pallas_call(
        body,
        grid=(nblk,),
        in_specs=[
            pl.BlockSpec((1, N_HEAD, blk, HEAD),
                         lambda i: (i // tb, 0, i % tb, 0)),
            pl.BlockSpec((blk, N_EMBD), lambda i: (i, 0)),
            pl.BlockSpec((N_EMBD, N_EMBD), lambda i: (0, 0)),
            pl.BlockSpec((1, N_EMBD), lambda i: (0, 0)),
            pl.BlockSpec((1, N_EMBD), lambda i: (0, 0)),
            pl.BlockSpec((E, N_EMBD), lambda i: (0, 0)),
            pl.BlockSpec((1, E), lambda i: (0, 0)),
        ],
        out_specs=(pl.BlockSpec((blk, N_EMBD), lambda i: (i, 0)),
                   pl.BlockSpec((blk, N_EMBD), lambda i: (i, 0)),
                   pl.BlockSpec((blk, 1), lambda i: (i, 0)),
                   pl.BlockSpec((blk, 1), lambda i: (i, 0)),
                   pl.BlockSpec((1, 1), lambda i: (0, 0))),
        out_shape=(jax.ShapeDtypeStruct((bt, N_EMBD), jnp.float32),
                   jax.ShapeDtypeStruct((bt, N_EMBD), jnp.float32),
                   jax.ShapeDtypeStruct((bt, 1), jnp.int32),
                   jax.ShapeDtypeStruct((bt, 1), jnp.float32),
                   jax.ShapeDtypeStruct((1, 1), jnp.float32)),
        scratch_shapes=[
            pltpu.VMEM((1, 128), jnp.float32),
            pltpu.VMEM((1, 128), jnp.float32),
            pltpu.VMEM((1, 128), jnp.float32),
        ],
    )(attnh, x2d, wo, g, b, wr, br)


def _sc_scatter(xn2, dest, nslot):
    bt, c = xn2.shape
    tpw = bt // SC_WORKERS

    @functools.partial(
        pl.kernel,
        out_type=jax.ShapeDtypeStruct((nslot, c), jnp.float32),
        mesh=_sc_mesh(),
        scratch_types=[
            pltpu.VMEM((tpw,), jnp.int32),
            pltpu.VMEM((tpw, c), jnp.float32),
            pltpu.SemaphoreType.DMA,
        ],
    )
    def k(xn_hbm, dest_hbm, buf_hbm, idx_v, rows_v, sem):
        wid = lax.axis_index("s") * SC_CORES + lax.axis_index("c")
        base = wid * tpw
        pltpu.sync_copy(dest_hbm.at[pl.ds(base, tpw)], idx_v)
        pltpu.sync_copy(xn_hbm.at[pl.ds(base, tpw)], rows_v)
        pltpu.async_copy(rows_v, buf_hbm.at[idx_v], sem).wait()

    return k(xn2, dest)


def _sc_gather(ybuf, dest, bt):
    c = ybuf.shape[1]
    tpw = bt // SC_WORKERS

    @functools.partial(
        pl.kernel,
        out_type=jax.ShapeDtypeStruct((bt, c), jnp.float32),
        mesh=_sc_mesh(),
        scratch_types=[
            pltpu.VMEM((tpw,), jnp.int32),
            pltpu.VMEM((tpw, c), jnp.float32),
            pltpu.SemaphoreType.DMA,
        ],
    )
    def k(ybuf_hbm, dest_hbm, out_hbm, idx_v, y_v, sem):
        wid = lax.axis_index("s") * SC_CORES + lax.axis_index("c")
        base = wid * tpw
        pltpu.sync_copy(dest_hbm.at[pl.ds(base, tpw)], idx_v)
        pltpu.async_copy(ybuf_hbm.at[idx_v], y_v, sem).wait()
        pltpu.sync_copy(y_v, out_hbm.at[pl.ds(base, tpw)])

    return k(ybuf, dest)


def _mlp_body(x_ref, w1_ref, w2_ref, y_ref):
    x = x_ref[...].astype(jnp.bfloat16)
    h = lax.dot_general(x, w1_ref[0], (((1,), (1,)), ((), ())),
                        preferred_element_type=jnp.float32)
    h = jnp.maximum(h, 0.0).astype(jnp.bfloat16)
    y_ref[...] = lax.dot_general(h, w2_ref[0], (((1,), (1,)), ((), ())),
                                 preferred_element_type=jnp.float32)


def _mlp(buf, w1, w2, cap, nslot, blk):
    nblk = cap // blk
    return pl.pallas_call(
        _mlp_body,
        grid=(E, nblk),
        in_specs=[
            pl.BlockSpec((blk, N_EMBD), lambda e, i: (e * nblk + i, 0)),
            pl.BlockSpec((1, HID, N_EMBD), lambda e, i: (e, 0, 0)),
            pl.BlockSpec((1, N_EMBD, HID), lambda e, i: (e, 0, 0)),
        ],
        out_specs=pl.BlockSpec((blk, N_EMBD), lambda e, i: (e * nblk + i, 0)),
        out_shape=jax.ShapeDtypeStruct((nslot, N_EMBD), jnp.float32),
    )(buf, w1, w2)


def _epi_body(h_ref, g_ref, y_ref, o_ref):
    g = g_ref[...]
    o_ref[...] = h_ref[...] + jnp.where(g > 0.0, g * y_ref[...], 0.0)


def _epilogue(h2d, gate, ygath, blk):
    bt = h2d.shape[0]
    return pl.pallas_call(
        _epi_body,
        grid=(bt // blk,),
        in_specs=[
            pl.BlockSpec((blk, N_EMBD), lambda i: (i, 0)),
            pl.BlockSpec((blk, 1), lambda i: (i, 0)),
            pl.BlockSpec((blk, N_EMBD), lambda i: (i, 0)),
        ],
        out_specs=pl.BlockSpec((blk, N_EMBD), lambda i: (i, 0)),
        out_shape=jax.ShapeDtypeStruct((bt, N_EMBD), jnp.float32),
    )(h2d, gate, ygath)


def kernel(x, adapter_id, params):
    p = params
    b, t, c = x.shape
    bt = b * t
    cap = int(math.ceil(CAP_F * bt / E))
    nslot = E * cap + SC_WORKERS
    aid = jnp.asarray(adapter_id).astype(jnp.int32)

    w_all = jnp.concatenate([p['Wq'], p['Wk'], p['Wv'], p['Wo']], axis=0)
    a_all = jnp.concatenate(
        [p['Aq'][aid], p['Ak'][aid], p['Av'][aid], p['Ao'][aid]], axis=0)
    b_bd = jnp.zeros((2 * NQ + 2 * NKV, 4 * R), jnp.float32)
    b_bd = b_bd.at[0:NQ, 0:R].set(p['Bq'][aid])
    b_bd = b_bd.at[NQ:NQ + NKV, R:2 * R].set(p['Bk'][aid])
    b_bd = b_bd.at[NQ + NKV:NQ + 2 * NKV, 2 * R:3 * R].set(p['Bv'][aid])
    b_bd = b_bd.at[NQ + 2 * NKV:, 3 * R:].set(p['Bo'][aid])
    qkvo = _fold(w_all[None], b_bd[None], a_all[None])[0]
    wqkv = qkvo[:QKV_W]
    wo = qkvo[QKV_W:]

    w1 = _fold(p['W1'], p['B1'][:, aid], p['A1'][:, aid],
               dtype=jnp.bfloat16)
    w2 = _fold(p['W2'], p['B2'][:, aid], p['A2'][:, aid],
               dtype=jnp.bfloat16)

    c_tab, s_tab = _rope_tables(t)
    x2d = x.reshape(bt, c)
    qkvh = _qkv(x2d, p['ln1_g'][None], p['ln1_b'][None], wqkv,
                c_tab, s_tab, b, t, blk=512)
    attnh = _flash(qkvh, blk=512)
    h2d, xn2, dest, gate, aux = _oproj_router(
        attnh, x2d, wo, p['ln2_g'][None], p['ln2_b'][None],
        p['Wr'], p['br'][None], cap, t, blk=512)

    buf = _sc_scatter(xn2, dest.reshape(bt), nslot)
    ybuf = _mlp(buf, w1, w2, cap, nslot, blk=1280)
    ygath = _sc_gather(ybuf, dest.reshape(bt), bt)
    out2d = _epilogue(h2d, gate, ygath, blk=512)
    return out2d.reshape(b, t, c), aux.reshape(())


# speedup vs baseline: 2.0857x; 1.2883x over previous
import functools
import math

import jax
import jax.numpy as jnp
from jax import lax
from jax.experimental import pallas as pl
from jax.experimental.pallas import tpu as pltpu
from jax.experimental.pallas import tpu_sc as plsc

N_EMBD = 384
N_HEAD = 8
N_KV = 2
HEAD = N_EMBD // N_HEAD
R = 4
E = 4
CAP_F = 1.25
LORA_SCALE = 1.0 / R
HID = 4 * N_EMBD
NQ = N_HEAD * HEAD
NKV = N_KV * HEAD
QKV_W = NQ + 2 * NKV

SC_CORES = 2
SC_SUBCORES = 16
SC_WORKERS = SC_CORES * SC_SUBCORES


def _sc_mesh():
    return plsc.VectorSubcoreMesh(
        core_axis_name="c", subcore_axis_name="s",
        num_cores=SC_CORES, num_subcores=SC_SUBCORES)


def _dot_t(x, w):
    return lax.dot_general(x, w, (((1,), (1,)), ((), ())),
                           preferred_element_type=jnp.float32)


def _fold_body(w_ref, b_ref, a_ref, o_ref):
    eff = w_ref[0] + LORA_SCALE * jnp.dot(
        b_ref[0], a_ref[0], preferred_element_type=jnp.float32)
    o_ref[0] = eff.astype(o_ref.dtype)


def _fold(w, b, a, dtype=jnp.float32):
    g, m, n = w.shape
    r = b.shape[-1]
    return pl.pallas_call(
        _fold_body,
        grid=(g,),
        in_specs=[
            pl.BlockSpec((1, m, n), lambda i: (i, 0, 0)),
            pl.BlockSpec((1, m, r), lambda i: (i, 0, 0)),
            pl.BlockSpec((1, r, n), lambda i: (i, 0, 0)),
        ],
        out_specs=pl.BlockSpec((1, m, n), lambda i: (i, 0, 0)),
        out_shape=jax.ShapeDtypeStruct((g, m, n), dtype),
    )(w, b, a)


def _rope_body(c_ref, s_ref):
    t = c_ref.shape[0]
    pos = lax.broadcasted_iota(jnp.int32, (t, HEAD // 2), 0).astype(jnp.float32)
    j = lax.broadcasted_iota(jnp.int32, (t, HEAD // 2), 1).astype(jnp.float32)
    ang = pos * jnp.exp(j * (-2.0 * math.log(10000.0) / HEAD))
    c24 = jnp.cos(ang)
    s24 = jnp.sin(ang)
    jr = lax.broadcasted_iota(jnp.int32, (HEAD // 2, QKV_W), 0)
    lc = lax.broadcasted_iota(jnp.int32, (HEAD // 2, QKV_W), 1)
    rot = lc < (NQ + NKV)
    sel = (((lc % HEAD) // 2) == jr) & rot
    m = sel.astype(jnp.float32)
    sgn = jnp.where((lc % 2) == 0, -1.0, 1.0)
    vlane = jnp.where(rot, 0.0, 1.0)[0:1]
    qscale = jnp.where(lc < NQ, math.log2(math.e) / math.sqrt(HEAD),
                       1.0)[0:1]
    c_full = jnp.dot(c24, m, preferred_element_type=jnp.float32) + vlane
    s_full = jnp.dot(s24, m * sgn, preferred_element_type=jnp.float32)
    c_ref[...] = c_full * qscale
    s_ref[...] = s_full * qscale


def _rope_tables(t):
    return pl.pallas_call(
        _rope_body,
        out_shape=(jax.ShapeDtypeStruct((t, QKV_W), jnp.float32),
                   jax.ShapeDtypeStruct((t, QKV_W), jnp.float32)),
    )()


def _qkv_body(x_ref, g_ref, b_ref, w_ref, c_ref, s_ref, o_ref):
    x = x_ref[...]
    m = jnp.mean(x, axis=-1, keepdims=True)
    v = jnp.mean((x - m) ** 2, axis=-1, keepdims=True)
    xn = (x - m) / jnp.sqrt(v + 1e-5) * g_ref[...] + b_ref[...]
    qkv = lax.dot_general(
        xn.astype(jnp.bfloat16), w_ref[...].astype(jnp.bfloat16),
        (((1,), (1,)), ((), ())), preferred_element_type=jnp.float32)
    lane = lax.broadcasted_iota(jnp.int32, qkv.shape, 1)
    even = (lane % 2) == 0
    nl = qkv.shape[1]
    rot = jnp.where(even, pltpu.roll(qkv, nl - 1, 1), pltpu.roll(qkv, 1, 1))
    qkv = (qkv * c_ref[...] + rot * s_ref[...]).astype(jnp.bfloat16)
    for j in range(QKV_W // HEAD):
        o_ref[0, j] = qkv[:, j * HEAD:(j + 1) * HEAD]


def _qkv(x2d, g, b, w, c_tab, s_tab, bb, t, blk):
    bt = x2d.shape[0]
    tb = t // blk
    nh = QKV_W // HEAD
    return pl.pallas_call(
        _qkv_body,
        grid=(bt // blk,),
        in_specs=[
            pl.BlockSpec((blk, N_EMBD), lambda i: (i, 0)),
            pl.BlockSpec((1, N_EMBD), lambda i: (0, 0)),
            pl.BlockSpec((1, N_EMBD), lambda i: (0, 0)),
            pl.BlockSpec((QKV_W, N_EMBD), lambda i: (0, 0)),
            pl.BlockSpec((blk, QKV_W), lambda i: (i % tb, 0)),
            pl.BlockSpec((blk, QKV_W), lambda i: (i % tb, 0)),
        ],
        out_specs=pl.BlockSpec((1, nh, blk, HEAD),
                               lambda i: (i // tb, 0, i % tb, 0)),
        out_shape=jax.ShapeDtypeStruct((bb, nh, t, HEAD), jnp.bfloat16),
    )(x2d, g, b, w, c_tab, s_tab)


def _tri_qk(pid, nq):
    qi = jnp.zeros((), jnp.int32)
    for q in range(1, nq):
        qi = qi + (pid >= (q * (q + 1)) // 2).astype(jnp.int32)
    ki = pid - qi * (qi + 1) // 2
    return qi, ki


def _flash_body(q_ref, k_ref, v_ref, o_ref, m_ref, l_ref, acc_ref,
                *, blk, nq, rep):
    pid = pl.program_id(2)
    qi, ki = _tri_qk(pid, nq)

    @pl.when(ki == 0)
    def _():
        m_ref[...] = jnp.full_like(m_ref, -1e30)
        l_ref[...] = jnp.zeros_like(l_ref)
        acc_ref[...] = jnp.zeros_like(acc_ref)

    def update(s):
        m_prev = m_ref[:, 0:1]
        m_cur = jnp.max(s, axis=-1, keepdims=True)
        m_new = jnp.maximum(m_prev, m_cur)
        alpha = jnp.exp2(m_prev - m_new)
        p = jnp.exp2(s - m_new)
        l_ref[:, 0:1] = l_ref[:, 0:1] * alpha + jnp.sum(p, axis=-1,
                                                        keepdims=True)
        m_ref[:, 0:1] = m_new
        acc_ref[...] = acc_ref[...] * alpha + jnp.dot(
            p.astype(jnp.bfloat16), v_ref[0, 0],
            preferred_element_type=jnp.float32)

    def scores():
        q = q_ref[0].reshape(rep * blk, HEAD)
        return lax.dot_general(q, k_ref[0, 0], (((1,), (1,)), ((), ())),
                               preferred_element_type=jnp.float32)

    @pl.when(ki < qi)
    def _():
        update(scores())

    @pl.when(ki == qi)
    def _():
        s = scores()
        r = lax.broadcasted_iota(jnp.int32, s.shape, 0)
        c = lax.broadcasted_iota(jnp.int32, s.shape, 1)
        update(jnp.where(c <= (r % blk), s, -1e30))
        o_ref[0] = (acc_ref[...] / l_ref[:, 0:1]).astype(
            jnp.bfloat16).reshape(rep, blk, HEAD)


def _flash(qkvh, blk):
    bb, _, t, d = qkvh.shape
    nq = t // blk
    npair = nq * (nq + 1) // 2
    rep = N_HEAD // N_KV
    kcol = NQ // HEAD
    vcol = (NQ + NKV) // HEAD

    def qmap(b_, g_, p_):
        qi, _ = _tri_qk(p_, nq)
        return (b_, g_, qi, 0)

    def kmap(b_, g_, p_):
        _, ki = _tri_qk(p_, nq)
        return (b_, kcol + g_, ki, 0)

    def vmap_(b_, g_, p_):
        _, ki = _tri_qk(p_, nq)
        return (b_, vcol + g_, ki, 0)

    return pl.pallas_call(
        functools.partial(_flash_body, blk=blk, nq=nq, rep=rep),
        grid=(bb, N_KV, npair),
        in_specs=[
            pl.BlockSpec((1, rep, blk, d), qmap),
            pl.BlockSpec((1, 1, blk, d), kmap),
            pl.BlockSpec((1, 1, blk, d), vmap_),
        ],
        out_specs=pl.BlockSpec((1, rep, blk, d), qmap),
        out_shape=jax.ShapeDtypeStruct((bb, N_HEAD, t, d), jnp.bfloat16),
        scratch_shapes=[
            pltpu.VMEM((rep * blk, 128), jnp.float32),
            pltpu.VMEM((rep * blk, 128), jnp.float32),
            pltpu.VMEM((rep * blk, d), jnp.float32),
        ],
    )(qkvh, qkvh, qkvh)


def _oproj_router_body(a_ref, x_ref, w_ref, g_ref, b_ref, wr_ref, br_ref,
                       h_ref, xn_ref, dest_ref, gate_ref, aux_ref,
                       cnt_ref, psum_ref, lsum_ref, *, blk, nblk, cap, bt):
    i = pl.program_id(0)

    @pl.when(i == 0)
    def _():
        cnt_ref[...] = jnp.zeros_like(cnt_ref)
        psum_ref[...] = jnp.zeros_like(psum_ref)
        lsum_ref[...] = jnp.zeros_like(lsum_ref)

    a = jnp.concatenate([a_ref[0, j] for j in range(N_HEAD)], axis=1)
    h = lax.dot_general(a, w_ref[...].astype(jnp.bfloat16),
                        (((1,), (1,)), ((), ())),
                        preferred_element_type=jnp.float32) + x_ref[...]
    h_ref[...] = h
    m = jnp.mean(h, axis=-1, keepdims=True)
    v = jnp.mean((h - m) ** 2, axis=-1, keepdims=True)
    xn = (h - m) / jnp.sqrt(v + 1e-5) * g_ref[...] + b_ref[...]
    xn_ref[...] = xn
    logits = _dot_t(xn, wr_ref[...]) + br_ref[...]
    mx = jnp.max(logits, axis=-1, keepdims=True)
    ex = jnp.exp(logits - mx)
    probs = ex / jnp.sum(ex, axis=-1, keepdims=True)
    top_v = jnp.max(probs, axis=-1, keepdims=True)
    lane = lax.broadcasted_iota(jnp.int32, probs.shape, 1)
    idx = jnp.min(jnp.where(probs >= top_v, lane, E), axis=-1,
                  keepdims=True)
    onehot = (lane == idx).astype(jnp.float32)
    row = lax.broadcasted_iota(jnp.int32, (blk, blk), 0)
    col = lax.broadcasted_iota(jnp.int32, (blk, blk), 1)
    tri = (row >= col).astype(jnp.float32)
    csum = jnp.dot(tri, onehot, preferred_element_type=jnp.float32)
    pos = csum - 1.0 + cnt_ref[0:1, 0:E]
    disp = onehot * (pos < cap).astype(jnp.float32)
    disp_tok = jnp.sum(disp, axis=-1, keepdims=True)
    pos_tok = jnp.sum(disp * pos, axis=-1, keepdims=True)
    rowid = i * blk + lax.broadcasted_iota(jnp.int32, (blk, 1), 0)
    dest_hit = idx * cap + pos_tok.astype(jnp.int32)
    trash = E * cap + rowid // (bt // SC_WORKERS)
    dest_ref[...] = jnp.where(disp_tok > 0.0, dest_hit, trash)
    gate_ref[...] = top_v * disp_tok
    cnt_ref[0:1, 0:E] = cnt_ref[0:1, 0:E] + jnp.sum(onehot, axis=0,
                                                    keepdims=True)
    psum_ref[0:1, 0:E] = psum_ref[0:1, 0:E] + jnp.sum(probs, axis=0,
                                                      keepdims=True)
    lsum_ref[0:1, 0:E] = lsum_ref[0:1, 0:E] + jnp.sum(disp, axis=0,
                                                      keepdims=True)

    @pl.when(i == nblk - 1)
    def _():
        prod = psum_ref[0:1, 0:E] * lsum_ref[0:1, 0:E]
        aux_ref[...] = jnp.sum(prod, axis=-1, keepdims=True) \
            * (float(E) / (float(bt) * float(bt)))


def _oproj_router(attnh, x2d, wo, g, b, wr, br, cap, t, blk):
    bt = x2d.shape[0]
    tb = t // blk
    nblk = bt // blk
    body = functools.partial(_oproj_router_body, blk=blk, nblk=nblk,
                             cap=cap, bt=bt)
    return pl.pallas_call(
        body,
        grid=(nblk,),
        in_specs=[
            pl.BlockSpec((1, N_HEAD, blk, HEAD),
                         lambda i: (i // tb, 0, i % tb, 0)),
            pl.BlockSpec((blk, N_EMBD), lambda i: (i, 0)),
            pl.BlockSpec((N_EMBD, N_EMBD), lambda i: (0, 0)),
            pl.BlockSpec((1, N_EMBD), lambda i: (0, 0)),
            pl.BlockSpec((1, N_EMBD), lambda i: (0, 0)),
            pl.BlockSpec((E, N_EMBD), lambda i: (0, 0)),
            pl.BlockSpec((1, E), lambda i: (0, 0)),
        ],
        out_specs=(pl.BlockSpec((blk, N_EMBD), lambda i: (i, 0)),
                   pl.BlockSpec((blk, N_EMBD), lambda i: (i, 0)),
                   pl.BlockSpec((blk, 1), lambda i: (i, 0)),
                   pl.BlockSpec((blk, 1), lambda i: (i, 0)),
                   pl.BlockSpec((1, 1), lambda i: (0, 0))),
        out_shape=(jax.ShapeDtypeStruct((bt, N_EMBD), jnp.float32),
                   jax.ShapeDtypeStruct((bt, N_EMBD), jnp.float32),
                   jax.ShapeDtypeStruct((bt, 1), jnp.int32),
                   jax.ShapeDtypeStruct((bt, 1), jnp.float32),
                   jax.ShapeDtypeStruct((1, 1), jnp.float32)),
        scratch_shapes=[
            pltpu.VMEM((1, 128), jnp.float32),
            pltpu.VMEM((1, 128), jnp.float32),
            pltpu.VMEM((1, 128), jnp.float32),
        ],
    )(attnh, x2d, wo, g, b, wr, br)


def _sc_scatter(xn2, dest, nslot):
    bt, c = xn2.shape
    tpw = bt // SC_WORKERS

    @functools.partial(
        pl.kernel,
        out_type=jax.ShapeDtypeStruct((nslot, c), jnp.float32),
        mesh=_sc_mesh(),
        scratch_types=[
            pltpu.VMEM((tpw,), jnp.int32),
            pltpu.VMEM((tpw, c), jnp.float32),
            pltpu.SemaphoreType.DMA,
        ],
    )
    def k(xn_hbm, dest_hbm, buf_hbm, idx_v, rows_v, sem):
        wid = lax.axis_index("s") * SC_CORES + lax.axis_index("c")
        base = wid * tpw
        pltpu.sync_copy(dest_hbm.at[pl.ds(base, tpw)], idx_v)
        pltpu.sync_copy(xn_hbm.at[pl.ds(base, tpw)], rows_v)
        pltpu.async_copy(rows_v, buf_hbm.at[idx_v], sem).wait()

    return k(xn2, dest)


def _sc_gather(ybuf, dest, bt):
    c = ybuf.shape[1]
    tpw = bt // SC_WORKERS

    @functools.partial(
        pl.kernel,
        out_type=jax.ShapeDtypeStruct((bt, c), jnp.float32),
        mesh=_sc_mesh(),
        scratch_types=[
            pltpu.VMEM((tpw,), jnp.int32),
            pltpu.VMEM((tpw, c), jnp.float32),
            pltpu.SemaphoreType.DMA,
        ],
    )
    def k(ybuf_hbm, dest_hbm, out_hbm, idx_v, y_v, sem):
        wid = lax.axis_index("s") * SC_CORES + lax.axis_index("c")
        base = wid * tpw
        pltpu.sync_copy(dest_hbm.at[pl.ds(base, tpw)], idx_v)
        pltpu.async_copy(ybuf_hbm.at[idx_v], y_v, sem).wait()
        pltpu.sync_copy(y_v, out_hbm.at[pl.ds(base, tpw)])

    return k(ybuf, dest)


def _mlp_body(x_ref, w1_ref, w2_ref, y_ref):
    x = x_ref[...].astype(jnp.bfloat16)
    h = lax.dot_general(x, w1_ref[0], (((1,), (1,)), ((), ())),
                        preferred_element_type=jnp.float32)
    h = jnp.maximum(h, 0.0).astype(jnp.bfloat16)
    y_ref[...] = lax.dot_general(h, w2_ref[0], (((1,), (1,)), ((), ())),
                                 preferred_element_type=jnp.float32)


def _mlp(buf, w1, w2, cap, nslot, blk):
    nblk = cap // blk
    return pl.pallas_call(
        _mlp_body,
        grid=(E, nblk),
        in_specs=[
            pl.BlockSpec((blk, N_EMBD), lambda e, i: (e * nblk + i, 0)),
            pl.BlockSpec((1, HID, N_EMBD), lambda e, i: (e, 0, 0)),
            pl.BlockSpec((1, N_EMBD, HID), lambda e, i: (e, 0, 0)),
        ],
        out_specs=pl.BlockSpec((blk, N_EMBD), lambda e, i: (e * nblk + i, 0)),
        out_shape=jax.ShapeDtypeStruct((nslot, N_EMBD), jnp.float32),
    )(buf, w1, w2)


def _epi_body(h_ref, g_ref, y_ref, o_ref):
    g = g_ref[...]
    o_ref[...] = h_ref[...] + jnp.where(g > 0.0, g * y_ref[...], 0.0)


def _epilogue(h2d, gate, ygath, blk):
    bt = h2d.shape[0]
    return pl.pallas_call(
        _epi_body,
        grid=(bt // blk,),
        in_specs=[
            pl.BlockSpec((blk, N_EMBD), lambda i: (i, 0)),
            pl.BlockSpec((blk, 1), lambda i: (i, 0)),
            pl.BlockSpec((blk, N_EMBD), lambda i: (i, 0)),
        ],
        out_specs=pl.BlockSpec((blk, N_EMBD), lambda i: (i, 0)),
        out_shape=jax.ShapeDtypeStruct((bt, N_EMBD), jnp.float32),
    )(h2d, gate, ygath)


def kernel(x, adapter_id, params):
    p = params
    b, t, c = x.shape
    bt = b * t
    cap = int(math.ceil(CAP_F * bt / E))
    nslot = E * cap + SC_WORKERS
    aid = jnp.asarray(adapter_id).astype(jnp.int32)

    w_all = jnp.concatenate([p['Wq'], p['Wk'], p['Wv'], p['Wo']], axis=0)
    a_all = jnp.concatenate(
        [p['Aq'][aid], p['Ak'][aid], p['Av'][aid], p['Ao'][aid]], axis=0)
    b_bd = jnp.zeros((2 * NQ + 2 * NKV, 4 * R), jnp.float32)
    b_bd = b_bd.at[0:NQ, 0:R].set(p['Bq'][aid])
    b_bd = b_bd.at[NQ:NQ + NKV, R:2 * R].set(p['Bk'][aid])
    b_bd = b_bd.at[NQ + NKV:NQ + 2 * NKV, 2 * R:3 * R].set(p['Bv'][aid])
    b_bd = b_bd.at[NQ + 2 * NKV:, 3 * R:].set(p['Bo'][aid])
    qkvo = _fold(w_all[None], b_bd[None], a_all[None])[0]
    wqkv = qkvo[:QKV_W]
    wo = qkvo[QKV_W:]

    w1 = _fold(p['W1'], p['B1'][:, aid], p['A1'][:, aid],
               dtype=jnp.bfloat16)
    w2 = _fold(p['W2'], p['B2'][:, aid], p['A2'][:, aid],
               dtype=jnp.bfloat16)

    c_tab, s_tab = _rope_tables(t)
    x2d = x.reshape(bt, c)
    qkvh = _qkv(x2d, p['ln1_g'][None], p['ln1_b'][None], wqkv,
                c_tab, s_tab, b, t, blk=512)
    attnh = _flash(qkvh, blk=512)
    h2d, xn2, dest, gate, aux = _oproj_router(
        attnh, x2d, wo, p['ln2_g'][None], p['ln2_b'][None],
        p['Wr'], p['br'][None], cap, t, blk=512)

    buf = _sc_scatter(xn2, dest.reshape(bt), nslot)
    ybuf = _mlp(buf, w1, w2, cap, nslot, blk=1280)
    ygath = _sc_gather(ybuf, dest.reshape(bt), bt)
    out2d = _epilogue(h2d, gate, ygath, blk=512)
    return out2d.reshape(b, t, c), aux.reshape(())
